# Initial kernel scaffold; baseline (speedup 1.0000x reference)
import jax
import jax.numpy as jnp
from jax.experimental import pallas as pl


def kernel(x, edge_index, edge_attr, energies, batch, emb, We1, We2, Wsk, Wf, Ws, Wfe1, Wfe2, Wfc1, Wfc2, be1, be2, bfe1, bfe2, bfc1, bfc2):
    raise NotImplementedError("write your pallas kernel here")


# speedup vs baseline: 1.7421x; 1.7421x over previous
import functools

import jax
import jax.numpy as jnp
from jax import lax
from jax.experimental import pallas as pl
from jax.experimental.pallas import tpu as pltpu
from jax.experimental.pallas import tpu_sc as plsc

N = 10000
E = 160000
G = 16
D = 256

NC = 2
NS = 16
NW = NC * NS

BN = 2000
BE = 2000
KG = 64
KS = 128
RW = 80
CW = N // RW


def _leaky(v):
    return jnp.where(v >= 0, v, 0.01 * v)


def _softplus(v):
    return jnp.maximum(v, 0.0) + jnp.log1p(jnp.exp(-jnp.abs(v)))


def _node0_body(x_ref, emb_ref, wd_ref, ws_ref, wsk_ref,
                td_ref, ts_ref, s0_ref, s1_ref):
    xb = x_ref[0, 0, :].reshape(BN, 1)
    oh = (xb == lax.broadcasted_iota(jnp.int32, (BN, 118), 1)).astype(jnp.float32)
    h = jnp.dot(oh, emb_ref[...], preferred_element_type=jnp.float32)
    td_ref[...] = jnp.dot(h, wd_ref[...], preferred_element_type=jnp.float32)
    ts_ref[...] = jnp.dot(h, ws_ref[...], preferred_element_type=jnp.float32)
    s = jnp.dot(h, wsk_ref[...], preferred_element_type=jnp.float32)
    s0_ref[...] = s[:, :128]
    s1_ref[...] = s[:, 128:]


def _node0_call(x3, emb, wd, ws, wsk):
    return pl.pallas_call(
        _node0_body,
        grid=(N // BN,),
        in_specs=[
            pl.BlockSpec((1, 1, BN), lambda i: (i, 0, 0)),
            pl.BlockSpec((118, D), lambda i: (0, 0)),
            pl.BlockSpec((D, 2 * D), lambda i: (0, 0)),
            pl.BlockSpec((D, 2 * D), lambda i: (0, 0)),
            pl.BlockSpec((D, D), lambda i: (0, 0)),
        ],
        out_specs=[
            pl.BlockSpec((BN, 2 * D), lambda i: (i, 0)),
            pl.BlockSpec((BN, 2 * D), lambda i: (i, 0)),
            pl.BlockSpec((BN, 128), lambda i: (i, 0)),
            pl.BlockSpec((BN, 128), lambda i: (i, 0)),
        ],
        out_shape=[
            jax.ShapeDtypeStruct((N, 2 * D), jnp.float32),
            jax.ShapeDtypeStruct((N, 2 * D), jnp.float32),
            jax.ShapeDtypeStruct((N, 128), jnp.float32),
            jax.ShapeDtypeStruct((N, 128), jnp.float32),
        ],
    )(x3, emb, wd, ws, wsk)


def _node12_body(a0_ref, a1_ref, p0_ref, p1_ref, wd_ref, ws_ref, wsk_ref,
                 td_ref, ts_ref, s0_ref, s1_ref):
    h = jnp.concatenate(
        [a0_ref[...] + p0_ref[...], a1_ref[...] + p1_ref[...]], axis=1)
    td_ref[...] = jnp.dot(h, wd_ref[...], preferred_element_type=jnp.float32)
    ts_ref[...] = jnp.dot(h, ws_ref[...], preferred_element_type=jnp.float32)
    s = jnp.dot(h, wsk_ref[...], preferred_element_type=jnp.float32)
    s0_ref[...] = s[:, :128]
    s1_ref[...] = s[:, 128:]


def _node12_call(a0, a1, p0, p1, wd, ws, wsk):
    half = pl.BlockSpec((BN, 128), lambda i: (i, 0))
    return pl.pallas_call(
        _node12_body,
        grid=(N // BN,),
        in_specs=[
            half, half, half, half,
            pl.BlockSpec((D, 2 * D), lambda i: (0, 0)),
            pl.BlockSpec((D, 2 * D), lambda i: (0, 0)),
            pl.BlockSpec((D, D), lambda i: (0, 0)),
        ],
        out_specs=[
            pl.BlockSpec((BN, 2 * D), lambda i: (i, 0)),
            pl.BlockSpec((BN, 2 * D), lambda i: (i, 0)),
            pl.BlockSpec((BN, 128), lambda i: (i, 0)),
            pl.BlockSpec((BN, 128), lambda i: (i, 0)),
        ],
        out_shape=[
            jax.ShapeDtypeStruct((N, 2 * D), jnp.float32),
            jax.ShapeDtypeStruct((N, 2 * D), jnp.float32),
            jax.ShapeDtypeStruct((N, 128), jnp.float32),
            jax.ShapeDtypeStruct((N, 128), jnp.float32),
        ],
    )(a0, a1, p0, p1, wd, ws, wsk)


def _edge_body(ea_ref, gd_ref, gs_ref, we1_ref, be1_ref, we2_ref, be2_ref,
               wedge_ref, m0_ref, m1_ref):
    e0 = jnp.dot(ea_ref[...], we1_ref[...],
                 preferred_element_type=jnp.float32) + be1_ref[...]
    e1 = jnp.dot(_leaky(e0), we2_ref[...],
                 preferred_element_type=jnp.float32) + be2_ref[...]
    pq = jnp.dot(e1, wedge_ref[...], preferred_element_type=jnp.float32)
    pq = pq + gd_ref[...] + gs_ref[...]
    p = pq[:, :D]
    q = pq[:, D:]
    m = (1.0 / (1.0 + jnp.exp(-p))) * _softplus(q)
    m0_ref[...] = m[:, :128]
    m1_ref[...] = m[:, 128:]


def _edge_call(edge_attr, gd, gs, we1, be1, we2, be2, wedge):
    return pl.pallas_call(
        _edge_body,
        grid=(E // BE,),
        in_specs=[
            pl.BlockSpec((BE, 14), lambda i: (i, 0)),
            pl.BlockSpec((BE, 2 * D), lambda i: (i, 0)),
            pl.BlockSpec((BE, 2 * D), lambda i: (i, 0)),
            pl.BlockSpec((14, 128), lambda i: (0, 0)),
            pl.BlockSpec((1, 128), lambda i: (0, 0)),
            pl.BlockSpec((128, D), lambda i: (0, 0)),
            pl.BlockSpec((1, D), lambda i: (0, 0)),
            pl.BlockSpec((D, 2 * D), lambda i: (0, 0)),
        ],
        out_specs=[
            pl.BlockSpec((BE, 128), lambda i: (i, 0)),
            pl.BlockSpec((BE, 128), lambda i: (i, 0)),
        ],
        out_shape=[
            jax.ShapeDtypeStruct((E, 128), jnp.float32),
            jax.ShapeDtypeStruct((E, 128), jnp.float32),
        ],
    )(edge_attr, gd, gs, we1, be1, we2, be2, wedge)


def _pool_body(a0_ref, a1_ref, p0_ref, p1_ref, b_ref,
               sum_ref, max_ref, cnt_ref):
    i = pl.program_id(0)

    @pl.when(i == 0)
    def _init():
        sum_ref[...] = jnp.zeros((G, D), jnp.float32)
        max_ref[...] = jnp.full((G, D), -jnp.inf, jnp.float32)
        cnt_ref[...] = jnp.zeros((G, 128), jnp.float32)

    h = jnp.concatenate(
        [a0_ref[...] + p0_ref[...], a1_ref[...] + p1_ref[...]], axis=1)
    bb = b_ref[0, 0, :].reshape(BN, 1)
    oh = (bb == lax.broadcasted_iota(jnp.int32, (BN, G), 1)).astype(jnp.float32)
    sum_ref[...] += lax.dot_general(
        oh, h, (((0,), (0,)), ((), ())), preferred_element_type=jnp.float32)
    cnt_ref[...] += jnp.broadcast_to(
        jnp.sum(oh, axis=0).reshape(G, 1), (G, 128))
    for g in range(G):
        sel = jnp.where(oh[:, g:g + 1] > 0, h, -jnp.inf)
        row = jnp.max(sel, axis=0).reshape(1, D)
        max_ref[g:g + 1, :] = jnp.maximum(max_ref[g:g + 1, :], row)


def _pool_call(a0, a1, p0, p1, b3):
    half = pl.BlockSpec((BN, 128), lambda i: (i, 0))
    return pl.pallas_call(
        _pool_body,
        grid=(N // BN,),
        in_specs=[
            half, half, half, half,
            pl.BlockSpec((1, 1, BN), lambda i: (i, 0, 0)),
        ],
        out_specs=[
            pl.BlockSpec((G, D), lambda i: (0, 0)),
            pl.BlockSpec((G, D), lambda i: (0, 0)),
            pl.BlockSpec((G, 128), lambda i: (0, 0)),
        ],
        out_shape=[
            jax.ShapeDtypeStruct((G, D), jnp.float32),
            jax.ShapeDtypeStruct((G, D), jnp.float32),
            jax.ShapeDtypeStruct((G, 128), jnp.float32),
        ],
    )(a0, a1, p0, p1, b3)


def _head_body(sum_ref, max_ref, cnt_ref, en_ref, wfe1_ref, bfe1_ref,
               wfe2_ref, bfe2_ref, wfc1_ref, bfc1_ref, wfc2_ref, bfc2_ref,
               out_ref):
    en = jnp.dot(en_ref[...], wfe1_ref[...],
                 preferred_element_type=jnp.float32) + bfe1_ref[...]
    en = jnp.dot(_leaky(en), wfe2_ref[...],
                 preferred_element_type=jnp.float32) + bfe2_ref[...]
    cnt = cnt_ref[...][:, 0:1]
    sump = sum_ref[...]
    meanp = sump / jnp.maximum(cnt, 1.0)
    crys = jnp.concatenate([meanp, max_ref[...], sump, en], axis=1)
    hid = jnp.dot(crys, wfc1_ref[...],
                  preferred_element_type=jnp.float32) + bfc1_ref[...]
    out_ref[...] = jnp.dot(_leaky(hid), wfc2_ref[...],
                           preferred_element_type=jnp.float32) + bfc2_ref[...]


def _head_call(sump, maxp, cnt, energies, wfe1, bfe1, wfe2, bfe2,
               wfc1, bfc1, wfc2, bfc2):
    full = lambda a: pl.BlockSpec(a.shape, lambda: tuple(0 for _ in a.shape))
    args = (sump, maxp, cnt, energies, wfe1, bfe1, wfe2, bfe2,
            wfc1, bfc1, wfc2, bfc2)
    return pl.pallas_call(
        _head_body,
        in_specs=[full(a) for a in args],
        out_specs=pl.BlockSpec((G, 804), lambda: (0, 0)),
        out_shape=jax.ShapeDtypeStruct((G, 804), jnp.float32),
    )(*args)


def _sc_mesh():
    return plsc.VectorSubcoreMesh(
        core_axis_name="c", subcore_axis_name="s",
        num_cores=NC, num_subcores=NS)


def _gather2_call(td, ts, dst, src):
    C = E // KG

    @functools.partial(
        pl.kernel,
        out_type=(jax.ShapeDtypeStruct((E, 2 * D), jnp.float32),
                  jax.ShapeDtypeStruct((E, 2 * D), jnp.float32)),
        mesh=_sc_mesh(),
        scratch_types=[
            pltpu.VMEM((KG,), jnp.int32),
            pltpu.VMEM((KG,), jnp.int32),
            pltpu.VMEM((KG, 2 * D), jnp.float32),
            pltpu.VMEM((KG, 2 * D), jnp.float32),
            pltpu.SemaphoreType.DMA,
            pltpu.SemaphoreType.DMA,
        ],
    )
    def k(td_h, ts_h, dst_h, src_h, gd_h, gs_h,
          idx_d, idx_s, buf_d, buf_s, sem0, sem1):
        wid = lax.axis_index("s") * NC + lax.axis_index("c")
        nloc = (C - wid + NW - 1) // NW

        def body(j, carry):
            c = wid + j * NW
            base = c * KG
            pltpu.sync_copy(dst_h.at[pl.ds(base, KG)], idx_d)
            pltpu.sync_copy(src_h.at[pl.ds(base, KG)], idx_s)
            cp0 = pltpu.async_copy(td_h.at[idx_d], buf_d, sem0)
            cp1 = pltpu.async_copy(ts_h.at[idx_s], buf_s, sem1)
            cp0.wait()
            cp1.wait()
            pltpu.sync_copy(buf_d, gd_h.at[pl.ds(base, KG)])
            pltpu.sync_copy(buf_s, gs_h.at[pl.ds(base, KG)])
            return carry

        lax.fori_loop(0, nloc, body, 0)

    return k(td, ts, dst, src)


def _scatter_call(m0, m1, dst):
    C = E // KS

    @functools.partial(
        pl.kernel,
        out_type=(jax.ShapeDtypeStruct((N, 128), jnp.float32),
                  jax.ShapeDtypeStruct((N, 128), jnp.float32)),
        mesh=_sc_mesh(),
        scratch_types=[
            pltpu.VMEM((KS,), jnp.int32),
            pltpu.VMEM((KS, 128), jnp.float32),
            pltpu.VMEM((RW, 128), jnp.float32),
            pltpu.VMEM_SHARED((N, 128), jnp.float32),
        ],
    )
    def k(m0_h, m1_h, dst_h, agg0_h, agg1_h, idxb, mbuf, obuf, acc):
        cid = lax.axis_index("c")
        sid = lax.axis_index("s")

        def zrow(r, carry):
            def zlane(j, c2):
                obuf[r, pl.ds(j * 16, 16)] = jnp.zeros((16,), jnp.float32)
                return c2
            return lax.fori_loop(0, 128 // 16, zlane, carry)

        lax.fori_loop(0, RW, zrow, 0)
        nw = (CW - sid + NS - 1) // NS

        def zchunk(j, carry):
            t = sid + j * NS
            pltpu.sync_copy(obuf, acc.at[pl.ds(t * RW, RW)])
            return carry

        lax.fori_loop(0, nw, zchunk, 0)
        plsc.subcore_barrier()

        nloc = (C - sid + NS - 1) // NS

        def body(j, carry):
            c = sid + j * NS
            base = c * KS
            pltpu.sync_copy(dst_h.at[pl.ds(base, KS)], idxb)

            @pl.when(cid == 0)
            def _l0():
                pltpu.sync_copy(m0_h.at[pl.ds(base, KS)], mbuf)

            @pl.when(cid == 1)
            def _l1():
                pltpu.sync_copy(m1_h.at[pl.ds(base, KS)], mbuf)

            pltpu.sync_copy(mbuf, acc.at[idxb], add=True)
            return carry

        lax.fori_loop(0, nloc, body, 0)
        plsc.subcore_barrier()

        def ochunk(j, carry):
            r0 = (sid + j * NS) * RW
            pltpu.sync_copy(acc.at[pl.ds(r0, RW)], obuf)

            @pl.when(cid == 0)
            def _s0():
                pltpu.sync_copy(obuf, agg0_h.at[pl.ds(r0, RW)])

            @pl.when(cid == 1)
            def _s1():
                pltpu.sync_copy(obuf, agg1_h.at[pl.ds(r0, RW)])

            return carry

        lax.fori_loop(0, nw, ochunk, 0)

    return k(m0, m1, dst)


def kernel(x, edge_index, edge_attr, energies, batch, emb, We1, We2, Wsk,
           Wf, Ws, Wfe1, Wfe2, Wfc1, Wfc2, be1, be2, bfe1, bfe2, bfc1, bfc2):
    src = edge_index[0].astype(jnp.int32)
    dst = edge_index[1].astype(jnp.int32)
    x3 = x.astype(jnp.int32).reshape(N // BN, 1, BN)
    b3 = batch.astype(jnp.int32).reshape(N // BN, 1, BN)

    wd = []
    wsrc = []
    wedge = []
    for i in range(3):
        wd.append(jnp.concatenate([Wf[i, :D, :], Ws[i, :D, :]], axis=1))
        wsrc.append(jnp.concatenate([Wf[i, D:2 * D, :], Ws[i, D:2 * D, :]], axis=1))
        wedge.append(jnp.concatenate([Wf[i, 2 * D:, :], Ws[i, 2 * D:, :]], axis=1))

    be1r = be1.reshape(1, 128)
    be2r = be2.reshape(1, D)

    td, ts, s0, s1 = _node0_call(x3, emb, wd[0], wsrc[0], Wsk[0])
    a0 = a1 = None
    for i in range(3):
        gd, gs = _gather2_call(td, ts, dst, src)
        m0, m1 = _edge_call(edge_attr, gd, gs, We1, be1r, We2, be2r, wedge[i])
        a0, a1 = _scatter_call(m0, m1, dst)
        if i < 2:
            td, ts, s0n, s1n = _node12_call(
                a0, a1, s0, s1, wd[i + 1], wsrc[i + 1], Wsk[i + 1])
            s0, s1 = s0n, s1n

    sump, maxp, cnt = _pool_call(a0, a1, s0, s1, b3)
    out = _head_call(
        sump, maxp, cnt, energies, Wfe1, bfe1.reshape(1, D),
        Wfe2, bfe2.reshape(1, 128), Wfc1, bfc1.reshape(1, 1024),
        Wfc2, bfc2.reshape(1, 804))
    return out.reshape(G, 4, 201)


# speedup vs baseline: 2.0610x; 1.1831x over previous
import functools

import jax
import jax.numpy as jnp
from jax import lax
from jax.experimental import pallas as pl
from jax.experimental.pallas import tpu as pltpu
from jax.experimental.pallas import tpu_sc as plsc

N = 10000
E = 160000
G = 16
D = 256

NC = 2
NS = 16
NW = NC * NS

BN = 2000
BE = 2000
KG = 40
KS = 128
RW = 80
CW = N // RW


def _leaky(v):
    return jnp.where(v >= 0, v, 0.01 * v)


def _softplus(v):
    return jnp.maximum(v, 0.0) + jnp.log1p(jnp.exp(-jnp.abs(v)))


def _node0_body(x_ref, emb_ref, wd_ref, ws_ref, wsk_ref,
                td_ref, ts_ref, s0_ref, s1_ref):
    xb = x_ref[0, 0, :].reshape(BN, 1)
    oh = (xb == lax.broadcasted_iota(jnp.int32, (BN, 118), 1)).astype(jnp.float32)
    h = jnp.dot(oh, emb_ref[...], preferred_element_type=jnp.float32)
    td_ref[...] = jnp.dot(h, wd_ref[...], preferred_element_type=jnp.float32)
    ts_ref[...] = jnp.dot(h, ws_ref[...], preferred_element_type=jnp.float32)
    s = jnp.dot(h, wsk_ref[...], preferred_element_type=jnp.float32)
    s0_ref[...] = s[:, :128]
    s1_ref[...] = s[:, 128:]


def _node0_call(x3, emb, wd, ws, wsk):
    return pl.pallas_call(
        _node0_body,
        grid=(N // BN,),
        in_specs=[
            pl.BlockSpec((1, 1, BN), lambda i: (i, 0, 0)),
            pl.BlockSpec((118, D), lambda i: (0, 0)),
            pl.BlockSpec((D, 2 * D), lambda i: (0, 0)),
            pl.BlockSpec((D, 2 * D), lambda i: (0, 0)),
            pl.BlockSpec((D, D), lambda i: (0, 0)),
        ],
        out_specs=[
            pl.BlockSpec((BN, 2 * D), lambda i: (i, 0)),
            pl.BlockSpec((BN, 2 * D), lambda i: (i, 0)),
            pl.BlockSpec((BN, 128), lambda i: (i, 0)),
            pl.BlockSpec((BN, 128), lambda i: (i, 0)),
        ],
        out_shape=[
            jax.ShapeDtypeStruct((N, 2 * D), jnp.float32),
            jax.ShapeDtypeStruct((N, 2 * D), jnp.float32),
            jax.ShapeDtypeStruct((N, 128), jnp.float32),
            jax.ShapeDtypeStruct((N, 128), jnp.float32),
        ],
    )(x3, emb, wd, ws, wsk)


def _node12_body(a0_ref, a1_ref, p0_ref, p1_ref, wd_ref, ws_ref, wsk_ref,
                 td_ref, ts_ref, s0_ref, s1_ref):
    h = jnp.concatenate(
        [a0_ref[...] + p0_ref[...], a1_ref[...] + p1_ref[...]], axis=1)
    td_ref[...] = jnp.dot(h, wd_ref[...], preferred_element_type=jnp.float32)
    ts_ref[...] = jnp.dot(h, ws_ref[...], preferred_element_type=jnp.float32)
    s = jnp.dot(h, wsk_ref[...], preferred_element_type=jnp.float32)
    s0_ref[...] = s[:, :128]
    s1_ref[...] = s[:, 128:]


def _node12_call(a0, a1, p0, p1, wd, ws, wsk):
    half = pl.BlockSpec((BN, 128), lambda i: (i, 0))
    return pl.pallas_call(
        _node12_body,
        grid=(N // BN,),
        in_specs=[
            half, half, half, half,
            pl.BlockSpec((D, 2 * D), lambda i: (0, 0)),
            pl.BlockSpec((D, 2 * D), lambda i: (0, 0)),
            pl.BlockSpec((D, D), lambda i: (0, 0)),
        ],
        out_specs=[
            pl.BlockSpec((BN, 2 * D), lambda i: (i, 0)),
            pl.BlockSpec((BN, 2 * D), lambda i: (i, 0)),
            pl.BlockSpec((BN, 128), lambda i: (i, 0)),
            pl.BlockSpec((BN, 128), lambda i: (i, 0)),
        ],
        out_shape=[
            jax.ShapeDtypeStruct((N, 2 * D), jnp.float32),
            jax.ShapeDtypeStruct((N, 2 * D), jnp.float32),
            jax.ShapeDtypeStruct((N, 128), jnp.float32),
            jax.ShapeDtypeStruct((N, 128), jnp.float32),
        ],
    )(a0, a1, p0, p1, wd, ws, wsk)


def _edge_body(ea_ref, g_ref, we1_ref, be1_ref, we2_ref, be2_ref,
               wedge_ref, m0_ref, m1_ref):
    e0 = jnp.dot(ea_ref[...], we1_ref[...],
                 preferred_element_type=jnp.float32) + be1_ref[...]
    e1 = jnp.dot(_leaky(e0), we2_ref[...],
                 preferred_element_type=jnp.float32) + be2_ref[...]
    pq = jnp.dot(e1, wedge_ref[...], preferred_element_type=jnp.float32)
    pq = pq + g_ref[...]
    p = pq[:, :D]
    q = pq[:, D:]
    m = (1.0 / (1.0 + jnp.exp(-p))) * _softplus(q)
    m0_ref[...] = m[:, :128]
    m1_ref[...] = m[:, 128:]


def _edge_call(edge_attr, g, we1, be1, we2, be2, wedge):
    return pl.pallas_call(
        _edge_body,
        grid=(E // BE,),
        in_specs=[
            pl.BlockSpec((BE, 14), lambda i: (i, 0)),
            pl.BlockSpec((BE, 2 * D), lambda i: (i, 0)),
            pl.BlockSpec((14, 128), lambda i: (0, 0)),
            pl.BlockSpec((1, 128), lambda i: (0, 0)),
            pl.BlockSpec((128, D), lambda i: (0, 0)),
            pl.BlockSpec((1, D), lambda i: (0, 0)),
            pl.BlockSpec((D, 2 * D), lambda i: (0, 0)),
        ],
        out_specs=[
            pl.BlockSpec((BE, 128), lambda i: (i, 0)),
            pl.BlockSpec((BE, 128), lambda i: (i, 0)),
        ],
        out_shape=[
            jax.ShapeDtypeStruct((E, 128), jnp.float32),
            jax.ShapeDtypeStruct((E, 128), jnp.float32),
        ],
    )(edge_attr, g, we1, be1, we2, be2, wedge)


def _pool_body(a0_ref, a1_ref, p0_ref, p1_ref, b_ref,
               sum_ref, max_ref, cnt_ref):
    i = pl.program_id(0)

    @pl.when(i == 0)
    def _init():
        sum_ref[...] = jnp.zeros((G, D), jnp.float32)
        max_ref[...] = jnp.full((G, D), -jnp.inf, jnp.float32)
        cnt_ref[...] = jnp.zeros((G, 128), jnp.float32)

    h = jnp.concatenate(
        [a0_ref[...] + p0_ref[...], a1_ref[...] + p1_ref[...]], axis=1)
    bb = b_ref[0, 0, :].reshape(BN, 1)
    oh = (bb == lax.broadcasted_iota(jnp.int32, (BN, G), 1)).astype(jnp.float32)
    sum_ref[...] += lax.dot_general(
        oh, h, (((0,), (0,)), ((), ())), preferred_element_type=jnp.float32)
    cnt_ref[...] += jnp.broadcast_to(
        jnp.sum(oh, axis=0).reshape(G, 1), (G, 128))
    for g in range(G):
        sel = jnp.where(oh[:, g:g + 1] > 0, h, -jnp.inf)
        row = jnp.max(sel, axis=0).reshape(1, D)
        max_ref[g:g + 1, :] = jnp.maximum(max_ref[g:g + 1, :], row)


def _pool_call(a0, a1, p0, p1, b3):
    half = pl.BlockSpec((BN, 128), lambda i: (i, 0))
    return pl.pallas_call(
        _pool_body,
        grid=(N // BN,),
        in_specs=[
            half, half, half, half,
            pl.BlockSpec((1, 1, BN), lambda i: (i, 0, 0)),
        ],
        out_specs=[
            pl.BlockSpec((G, D), lambda i: (0, 0)),
            pl.BlockSpec((G, D), lambda i: (0, 0)),
            pl.BlockSpec((G, 128), lambda i: (0, 0)),
        ],
        out_shape=[
            jax.ShapeDtypeStruct((G, D), jnp.float32),
            jax.ShapeDtypeStruct((G, D), jnp.float32),
            jax.ShapeDtypeStruct((G, 128), jnp.float32),
        ],
    )(a0, a1, p0, p1, b3)


def _head_body(sum_ref, max_ref, cnt_ref, en_ref, wfe1_ref, bfe1_ref,
               wfe2_ref, bfe2_ref, wfc1_ref, bfc1_ref, wfc2_ref, bfc2_ref,
               out_ref):
    en = jnp.dot(en_ref[...], wfe1_ref[...],
                 preferred_element_type=jnp.float32) + bfe1_ref[...]
    en = jnp.dot(_leaky(en), wfe2_ref[...],
                 preferred_element_type=jnp.float32) + bfe2_ref[...]
    cnt = cnt_ref[...][:, 0:1]
    sump = sum_ref[...]
    meanp = sump / jnp.maximum(cnt, 1.0)
    crys = jnp.concatenate([meanp, max_ref[...], sump, en], axis=1)
    hid = jnp.dot(crys, wfc1_ref[...],
                  preferred_element_type=jnp.float32) + bfc1_ref[...]
    out_ref[...] = jnp.dot(_leaky(hid), wfc2_ref[...],
                           preferred_element_type=jnp.float32) + bfc2_ref[...]


def _head_call(sump, maxp, cnt, energies, wfe1, bfe1, wfe2, bfe2,
               wfc1, bfc1, wfc2, bfc2):
    full = lambda a: pl.BlockSpec(a.shape, lambda: tuple(0 for _ in a.shape))
    args = (sump, maxp, cnt, energies, wfe1, bfe1, wfe2, bfe2,
            wfc1, bfc1, wfc2, bfc2)
    return pl.pallas_call(
        _head_body,
        in_specs=[full(a) for a in args],
        out_specs=pl.BlockSpec((G, 804), lambda: (0, 0)),
        out_shape=jax.ShapeDtypeStruct((G, 804), jnp.float32),
    )(*args)


def _sc_mesh():
    return plsc.VectorSubcoreMesh(
        core_axis_name="c", subcore_axis_name="s",
        num_cores=NC, num_subcores=NS)


def _gatheradd_call(td, ts, dst, src):
    C = E // KG

    @functools.partial(
        pl.kernel,
        out_type=jax.ShapeDtypeStruct((E, 2 * D), jnp.float32),
        mesh=_sc_mesh(),
        scratch_types=[
            pltpu.VMEM((KG,), jnp.int32),
            pltpu.VMEM((KG,), jnp.int32),
            pltpu.VMEM((KG,), jnp.int32),
            pltpu.VMEM((KG,), jnp.int32),
            pltpu.VMEM((KG, 2 * D), jnp.float32),
            pltpu.VMEM((KG, 2 * D), jnp.float32),
            pltpu.VMEM((KG, 2 * D), jnp.float32),
            pltpu.VMEM((KG, 2 * D), jnp.float32),
            pltpu.SemaphoreType.DMA,
            pltpu.SemaphoreType.DMA,
            pltpu.SemaphoreType.DMA,
            pltpu.SemaphoreType.DMA,
        ],
    )
    def k(td_h, ts_h, dst_h, src_h, g_h, i_d0, i_s0, i_d1, i_s1,
          bd0, bs0, bd1, bs1, gsem0, gsem1, wsem0, wsem1):
        wid = lax.axis_index("s") * NC + lax.axis_index("c")
        nloc = (C - wid + NW - 1) // NW
        idx = ((i_d0, i_s0), (i_d1, i_s1))
        bufs = ((bd0, bs0), (bd1, bs1))
        gsems = (gsem0, gsem1)
        wsems = (wsem0, wsem1)

        def base_of(j):
            return (wid + j * NW) * KG

        def stage_and_fire(j, slot):
            base = base_of(j)
            pltpu.sync_copy(dst_h.at[pl.ds(base, KG)], idx[slot][0])
            pltpu.sync_copy(src_h.at[pl.ds(base, KG)], idx[slot][1])
            pltpu.async_copy(td_h.at[idx[slot][0]], bufs[slot][0], gsems[slot])
            pltpu.async_copy(ts_h.at[idx[slot][1]], bufs[slot][1], gsems[slot])

        def wait_gathers(slot):
            pltpu.make_async_copy(
                td_h.at[idx[slot][0]], bufs[slot][0], gsems[slot]).wait()
            pltpu.make_async_copy(
                ts_h.at[idx[slot][1]], bufs[slot][1], gsems[slot]).wait()

        def drain_writeout(j, slot):
            pltpu.make_async_copy(
                bufs[slot][0], g_h.at[pl.ds(base_of(j), KG)],
                wsems[slot]).wait()

        stage_and_fire(0, 0)

        def pair(j2, carry):
            for b in range(2):
                j = j2 * 2 + b
                slot = b
                other = 1 - b

                @pl.when(j < nloc)
                def _step():
                    wait_gathers(slot)

                    @pl.when(j + 1 < nloc)
                    def _fire_next():
                        @pl.when(j >= 1)
                        def _drain_prev():
                            drain_writeout(j - 1, other)

                        stage_and_fire(j + 1, other)

                    bd, bs = bufs[slot]

                    @plsc.parallel_loop(0, KG)
                    def _add(r):
                        for t in range(2 * D // 16):
                            sl = pl.ds(t * 16, 16)
                            bd[r, sl] = bd[r, sl] + bs[r, sl]

                    pltpu.async_copy(
                        bufs[slot][0], g_h.at[pl.ds(base_of(j), KG)],
                        wsems[slot])
            return carry

        lax.fori_loop(0, (nloc + 1) // 2, pair, 0)

        last_even = (nloc - 1) % 2 == 0

        @pl.when((nloc >= 1) & last_even)
        def _drain_a():
            drain_writeout(nloc - 1, 0)

        @pl.when((nloc >= 1) & jnp.logical_not(last_even))
        def _drain_b():
            drain_writeout(nloc - 1, 1)

        @pl.when((nloc >= 2) & last_even)
        def _drain_c():
            drain_writeout(nloc - 2, 1)

        @pl.when((nloc >= 2) & jnp.logical_not(last_even))
        def _drain_d():
            drain_writeout(nloc - 2, 0)

    return k(td, ts, dst, src)


def _scatter_call(m0, m1, dst):
    C = E // KS

    @functools.partial(
        pl.kernel,
        out_type=(jax.ShapeDtypeStruct((N, 128), jnp.float32),
                  jax.ShapeDtypeStruct((N, 128), jnp.float32)),
        mesh=_sc_mesh(),
        scratch_types=[
            pltpu.VMEM((KS,), jnp.int32),
            pltpu.VMEM((KS, 128), jnp.float32),
            pltpu.VMEM((RW, 128), jnp.float32),
            pltpu.VMEM_SHARED((N, 128), jnp.float32),
        ],
    )
    def k(m0_h, m1_h, dst_h, agg0_h, agg1_h, idxb, mbuf, obuf, acc):
        cid = lax.axis_index("c")
        sid = lax.axis_index("s")

        def zrow(r, carry):
            def zlane(j, c2):
                obuf[r, pl.ds(j * 16, 16)] = jnp.zeros((16,), jnp.float32)
                return c2
            return lax.fori_loop(0, 128 // 16, zlane, carry)

        lax.fori_loop(0, RW, zrow, 0)
        nw = (CW - sid + NS - 1) // NS

        def zchunk(j, carry):
            t = sid + j * NS
            pltpu.sync_copy(obuf, acc.at[pl.ds(t * RW, RW)])
            return carry

        lax.fori_loop(0, nw, zchunk, 0)
        plsc.subcore_barrier()

        nloc = (C - sid + NS - 1) // NS

        def body(j, carry):
            c = sid + j * NS
            base = c * KS
            pltpu.sync_copy(dst_h.at[pl.ds(base, KS)], idxb)

            @pl.when(cid == 0)
            def _l0():
                pltpu.sync_copy(m0_h.at[pl.ds(base, KS)], mbuf)

            @pl.when(cid == 1)
            def _l1():
                pltpu.sync_copy(m1_h.at[pl.ds(base, KS)], mbuf)

            pltpu.sync_copy(mbuf, acc.at[idxb], add=True)
            return carry

        lax.fori_loop(0, nloc, body, 0)
        plsc.subcore_barrier()

        def ochunk(j, carry):
            r0 = (sid + j * NS) * RW
            pltpu.sync_copy(acc.at[pl.ds(r0, RW)], obuf)

            @pl.when(cid == 0)
            def _s0():
                pltpu.sync_copy(obuf, agg0_h.at[pl.ds(r0, RW)])

            @pl.when(cid == 1)
            def _s1():
                pltpu.sync_copy(obuf, agg1_h.at[pl.ds(r0, RW)])

            return carry

        lax.fori_loop(0, nw, ochunk, 0)

    return k(m0, m1, dst)


def kernel(x, edge_index, edge_attr, energies, batch, emb, We1, We2, Wsk,
           Wf, Ws, Wfe1, Wfe2, Wfc1, Wfc2, be1, be2, bfe1, bfe2, bfc1, bfc2):
    src = edge_index[0].astype(jnp.int32)
    dst = edge_index[1].astype(jnp.int32)
    x3 = x.astype(jnp.int32).reshape(N // BN, 1, BN)
    b3 = batch.astype(jnp.int32).reshape(N // BN, 1, BN)

    wd = []
    wsrc = []
    wedge = []
    for i in range(3):
        wd.append(jnp.concatenate([Wf[i, :D, :], Ws[i, :D, :]], axis=1))
        wsrc.append(jnp.concatenate([Wf[i, D:2 * D, :], Ws[i, D:2 * D, :]], axis=1))
        wedge.append(jnp.concatenate([Wf[i, 2 * D:, :], Ws[i, 2 * D:, :]], axis=1))

    be1r = be1.reshape(1, 128)
    be2r = be2.reshape(1, D)

    td, ts, s0, s1 = _node0_call(x3, emb, wd[0], wsrc[0], Wsk[0])
    a0 = a1 = None
    for i in range(3):
        g = _gatheradd_call(td, ts, dst, src)
        m0, m1 = _edge_call(edge_attr, g, We1, be1r, We2, be2r, wedge[i])
        a0, a1 = _scatter_call(m0, m1, dst)
        if i < 2:
            td, ts, s0n, s1n = _node12_call(
                a0, a1, s0, s1, wd[i + 1], wsrc[i + 1], Wsk[i + 1])
            s0, s1 = s0n, s1n

    sump, maxp, cnt = _pool_call(a0, a1, s0, s1, b3)
    out = _head_call(
        sump, maxp, cnt, energies, Wfe1, bfe1.reshape(1, D),
        Wfe2, bfe2.reshape(1, 128), Wfc1, bfc1.reshape(1, 1024),
        Wfc2, bfc2.reshape(1, 804))
    return out.reshape(G, 4, 201)


# speedup vs baseline: 2.2945x; 1.1133x over previous
import functools

import jax
import jax.numpy as jnp
from jax import lax
from jax.experimental import pallas as pl
from jax.experimental.pallas import tpu as pltpu
from jax.experimental.pallas import tpu_sc as plsc

N = 10000
E = 160000
G = 16
D = 256

NC = 2
NS = 16
NW = NC * NS

BN = 2000
BE = 2000
KG = 40
KS = 128
RW = 80
CW = N // RW


def _leaky(v):
    return jnp.where(v >= 0, v, 0.01 * v)


def _softplus(v):
    return jnp.maximum(v, 0.0) + jnp.log1p(jnp.exp(-jnp.abs(v)))


def _node0_body(x_ref, emb_ref, wd_ref, ws_ref, wsk_ref,
                td_ref, ts_ref, s0_ref, s1_ref):
    xb = x_ref[0, 0, :].reshape(BN, 1)
    oh = (xb == lax.broadcasted_iota(jnp.int32, (BN, 118), 1)).astype(jnp.float32)
    h = jnp.dot(oh, emb_ref[...], preferred_element_type=jnp.float32)
    td_ref[...] = jnp.dot(h, wd_ref[...], preferred_element_type=jnp.float32)
    ts_ref[...] = jnp.dot(h, ws_ref[...], preferred_element_type=jnp.float32)
    s = jnp.dot(h, wsk_ref[...], preferred_element_type=jnp.float32)
    s0_ref[...] = s[:, :128]
    s1_ref[...] = s[:, 128:]


def _node0_call(x3, emb, wd, ws, wsk):
    return pl.pallas_call(
        _node0_body,
        grid=(N // BN,),
        in_specs=[
            pl.BlockSpec((1, 1, BN), lambda i: (i, 0, 0)),
            pl.BlockSpec((118, D), lambda i: (0, 0)),
            pl.BlockSpec((D, 2 * D), lambda i: (0, 0)),
            pl.BlockSpec((D, 2 * D), lambda i: (0, 0)),
            pl.BlockSpec((D, D), lambda i: (0, 0)),
        ],
        out_specs=[
            pl.BlockSpec((BN, 2 * D), lambda i: (i, 0)),
            pl.BlockSpec((BN, 2 * D), lambda i: (i, 0)),
            pl.BlockSpec((BN, 128), lambda i: (i, 0)),
            pl.BlockSpec((BN, 128), lambda i: (i, 0)),
        ],
        out_shape=[
            jax.ShapeDtypeStruct((N, 2 * D), jnp.float32),
            jax.ShapeDtypeStruct((N, 2 * D), jnp.float32),
            jax.ShapeDtypeStruct((N, 128), jnp.float32),
            jax.ShapeDtypeStruct((N, 128), jnp.float32),
        ],
    )(x3, emb, wd, ws, wsk)


def _node12_body(a0a_ref, a1a_ref, a0b_ref, a1b_ref, p0_ref, p1_ref,
                 wd_ref, ws_ref, wsk_ref,
                 td_ref, ts_ref, s0_ref, s1_ref):
    h = jnp.concatenate(
        [a0a_ref[...] + a0b_ref[...] + p0_ref[...],
         a1a_ref[...] + a1b_ref[...] + p1_ref[...]], axis=1)
    td_ref[...] = jnp.dot(h, wd_ref[...], preferred_element_type=jnp.float32)
    ts_ref[...] = jnp.dot(h, ws_ref[...], preferred_element_type=jnp.float32)
    s = jnp.dot(h, wsk_ref[...], preferred_element_type=jnp.float32)
    s0_ref[...] = s[:, :128]
    s1_ref[...] = s[:, 128:]


def _node12_call(a0a, a1a, a0b, a1b, p0, p1, wd, ws, wsk):
    half = pl.BlockSpec((BN, 128), lambda i: (i, 0))
    return pl.pallas_call(
        _node12_body,
        grid=(N // BN,),
        in_specs=[
            half, half, half, half, half, half,
            pl.BlockSpec((D, 2 * D), lambda i: (0, 0)),
            pl.BlockSpec((D, 2 * D), lambda i: (0, 0)),
            pl.BlockSpec((D, D), lambda i: (0, 0)),
        ],
        out_specs=[
            pl.BlockSpec((BN, 2 * D), lambda i: (i, 0)),
            pl.BlockSpec((BN, 2 * D), lambda i: (i, 0)),
            pl.BlockSpec((BN, 128), lambda i: (i, 0)),
            pl.BlockSpec((BN, 128), lambda i: (i, 0)),
        ],
        out_shape=[
            jax.ShapeDtypeStruct((N, 2 * D), jnp.float32),
            jax.ShapeDtypeStruct((N, 2 * D), jnp.float32),
            jax.ShapeDtypeStruct((N, 128), jnp.float32),
            jax.ShapeDtypeStruct((N, 128), jnp.float32),
        ],
    )(a0a, a1a, a0b, a1b, p0, p1, wd, ws, wsk)


def _edge_body(ea_ref, g_ref, we1_ref, be1_ref, we2_ref, be2_ref,
               wedge_ref, m0_ref, m1_ref):
    e0 = jnp.dot(ea_ref[...], we1_ref[...],
                 preferred_element_type=jnp.float32) + be1_ref[...]
    e1 = jnp.dot(_leaky(e0), we2_ref[...],
                 preferred_element_type=jnp.float32) + be2_ref[...]
    pq = jnp.dot(e1, wedge_ref[...], preferred_element_type=jnp.float32)
    pq = pq + g_ref[...]
    p = pq[:, :D]
    q = pq[:, D:]
    m = (1.0 / (1.0 + jnp.exp(-p))) * _softplus(q)
    m0_ref[...] = m[:, :128]
    m1_ref[...] = m[:, 128:]


def _edge_call(edge_attr, g, we1, be1, we2, be2, wedge):
    ne = edge_attr.shape[0]
    return pl.pallas_call(
        _edge_body,
        grid=(ne // BE,),
        in_specs=[
            pl.BlockSpec((BE, 14), lambda i: (i, 0)),
            pl.BlockSpec((BE, 2 * D), lambda i: (i, 0)),
            pl.BlockSpec((14, 128), lambda i: (0, 0)),
            pl.BlockSpec((1, 128), lambda i: (0, 0)),
            pl.BlockSpec((128, D), lambda i: (0, 0)),
            pl.BlockSpec((1, D), lambda i: (0, 0)),
            pl.BlockSpec((D, 2 * D), lambda i: (0, 0)),
        ],
        out_specs=[
            pl.BlockSpec((BE, 128), lambda i: (i, 0)),
            pl.BlockSpec((BE, 128), lambda i: (i, 0)),
        ],
        out_shape=[
            jax.ShapeDtypeStruct((ne, 128), jnp.float32),
            jax.ShapeDtypeStruct((ne, 128), jnp.float32),
        ],
    )(edge_attr, g, we1, be1, we2, be2, wedge)


def _pool_body(a0a_ref, a1a_ref, a0b_ref, a1b_ref, p0_ref, p1_ref, b_ref,
               sum_ref, max_ref, cnt_ref):
    i = pl.program_id(0)

    @pl.when(i == 0)
    def _init():
        sum_ref[...] = jnp.zeros((G, D), jnp.float32)
        max_ref[...] = jnp.full((G, D), -jnp.inf, jnp.float32)
        cnt_ref[...] = jnp.zeros((G, 128), jnp.float32)

    h = jnp.concatenate(
        [a0a_ref[...] + a0b_ref[...] + p0_ref[...],
         a1a_ref[...] + a1b_ref[...] + p1_ref[...]], axis=1)
    bb = b_ref[0, 0, :].reshape(BN, 1)
    oh = (bb == lax.broadcasted_iota(jnp.int32, (BN, G), 1)).astype(jnp.float32)
    sum_ref[...] += lax.dot_general(
        oh, h, (((0,), (0,)), ((), ())), preferred_element_type=jnp.float32)
    cnt_ref[...] += jnp.broadcast_to(
        jnp.sum(oh, axis=0).reshape(G, 1), (G, 128))
    for g in range(G):
        sel = jnp.where(oh[:, g:g + 1] > 0, h, -jnp.inf)
        row = jnp.max(sel, axis=0).reshape(1, D)
        max_ref[g:g + 1, :] = jnp.maximum(max_ref[g:g + 1, :], row)


def _pool_call(a0a, a1a, a0b, a1b, p0, p1, b3):
    half = pl.BlockSpec((BN, 128), lambda i: (i, 0))
    return pl.pallas_call(
        _pool_body,
        grid=(N // BN,),
        in_specs=[
            half, half, half, half, half, half,
            pl.BlockSpec((1, 1, BN), lambda i: (i, 0, 0)),
        ],
        out_specs=[
            pl.BlockSpec((G, D), lambda i: (0, 0)),
            pl.BlockSpec((G, D), lambda i: (0, 0)),
            pl.BlockSpec((G, 128), lambda i: (0, 0)),
        ],
        out_shape=[
            jax.ShapeDtypeStruct((G, D), jnp.float32),
            jax.ShapeDtypeStruct((G, D), jnp.float32),
            jax.ShapeDtypeStruct((G, 128), jnp.float32),
        ],
    )(a0a, a1a, a0b, a1b, p0, p1, b3)


def _head_body(sum_ref, max_ref, cnt_ref, en_ref, wfe1_ref, bfe1_ref,
               wfe2_ref, bfe2_ref, wfc1_ref, bfc1_ref, wfc2_ref, bfc2_ref,
               out_ref):
    en = jnp.dot(en_ref[...], wfe1_ref[...],
                 preferred_element_type=jnp.float32) + bfe1_ref[...]
    en = jnp.dot(_leaky(en), wfe2_ref[...],
                 preferred_element_type=jnp.float32) + bfe2_ref[...]
    cnt = cnt_ref[...][:, 0:1]
    sump = sum_ref[...]
    meanp = sump / jnp.maximum(cnt, 1.0)
    crys = jnp.concatenate([meanp, max_ref[...], sump, en], axis=1)
    hid = jnp.dot(crys, wfc1_ref[...],
                  preferred_element_type=jnp.float32) + bfc1_ref[...]
    out_ref[...] = jnp.dot(_leaky(hid), wfc2_ref[...],
                           preferred_element_type=jnp.float32) + bfc2_ref[...]


def _head_call(sump, maxp, cnt, energies, wfe1, bfe1, wfe2, bfe2,
               wfc1, bfc1, wfc2, bfc2):
    full = lambda a: pl.BlockSpec(a.shape, lambda: tuple(0 for _ in a.shape))
    args = (sump, maxp, cnt, energies, wfe1, bfe1, wfe2, bfe2,
            wfc1, bfc1, wfc2, bfc2)
    return pl.pallas_call(
        _head_body,
        in_specs=[full(a) for a in args],
        out_specs=pl.BlockSpec((G, 804), lambda: (0, 0)),
        out_shape=jax.ShapeDtypeStruct((G, 804), jnp.float32),
    )(*args)


def _sc_mesh():
    return plsc.VectorSubcoreMesh(
        core_axis_name="c", subcore_axis_name="s",
        num_cores=NC, num_subcores=NS)


def _gatheradd_call(td, ts, dst, src):
    ne = dst.shape[0]
    C = ne // KG

    @functools.partial(
        pl.kernel,
        out_type=jax.ShapeDtypeStruct((ne, 2 * D), jnp.float32),
        mesh=_sc_mesh(),
        scratch_types=[
            pltpu.VMEM((KG,), jnp.int32),
            pltpu.VMEM((KG,), jnp.int32),
            pltpu.VMEM((KG,), jnp.int32),
            pltpu.VMEM((KG,), jnp.int32),
            pltpu.VMEM((KG, 2 * D), jnp.float32),
            pltpu.VMEM((KG, 2 * D), jnp.float32),
            pltpu.VMEM((KG, 2 * D), jnp.float32),
            pltpu.VMEM((KG, 2 * D), jnp.float32),
            pltpu.SemaphoreType.DMA,
            pltpu.SemaphoreType.DMA,
            pltpu.SemaphoreType.DMA,
            pltpu.SemaphoreType.DMA,
        ],
    )
    def k(td_h, ts_h, dst_h, src_h, g_h, i_d0, i_s0, i_d1, i_s1,
          bd0, bs0, bd1, bs1, gsem0, gsem1, wsem0, wsem1):
        wid = lax.axis_index("s") * NC + lax.axis_index("c")
        nloc = (C - wid + NW - 1) // NW
        idx = ((i_d0, i_s0), (i_d1, i_s1))
        bufs = ((bd0, bs0), (bd1, bs1))
        gsems = (gsem0, gsem1)
        wsems = (wsem0, wsem1)

        def base_of(j):
            return (wid + j * NW) * KG

        def stage_and_fire(j, slot):
            base = base_of(j)
            pltpu.sync_copy(dst_h.at[pl.ds(base, KG)], idx[slot][0])
            pltpu.sync_copy(src_h.at[pl.ds(base, KG)], idx[slot][1])
            pltpu.async_copy(td_h.at[idx[slot][0]], bufs[slot][0], gsems[slot])
            pltpu.async_copy(ts_h.at[idx[slot][1]], bufs[slot][1], gsems[slot])

        def wait_gathers(slot):
            pltpu.make_async_copy(
                td_h.at[idx[slot][0]], bufs[slot][0], gsems[slot]).wait()
            pltpu.make_async_copy(
                ts_h.at[idx[slot][1]], bufs[slot][1], gsems[slot]).wait()

        def drain_writeout(j, slot):
            pltpu.make_async_copy(
                bufs[slot][0], g_h.at[pl.ds(base_of(j), KG)],
                wsems[slot]).wait()

        stage_and_fire(0, 0)

        def pair(j2, carry):
            for b in range(2):
                j = j2 * 2 + b
                slot = b
                other = 1 - b

                @pl.when(j < nloc)
                def _step():
                    wait_gathers(slot)

                    @pl.when(j + 1 < nloc)
                    def _fire_next():
                        @pl.when(j >= 1)
                        def _drain_prev():
                            drain_writeout(j - 1, other)

                        stage_and_fire(j + 1, other)

                    bd, bs = bufs[slot]

                    @plsc.parallel_loop(0, KG)
                    def _add(r):
                        for t in range(2 * D // 16):
                            sl = pl.ds(t * 16, 16)
                            bd[r, sl] = bd[r, sl] + bs[r, sl]

                    pltpu.async_copy(
                        bufs[slot][0], g_h.at[pl.ds(base_of(j), KG)],
                        wsems[slot])
            return carry

        lax.fori_loop(0, (nloc + 1) // 2, pair, 0)

        last_even = (nloc - 1) % 2 == 0

        @pl.when((nloc >= 1) & last_even)
        def _drain_a():
            drain_writeout(nloc - 1, 0)

        @pl.when((nloc >= 1) & jnp.logical_not(last_even))
        def _drain_b():
            drain_writeout(nloc - 1, 1)

        @pl.when((nloc >= 2) & last_even)
        def _drain_c():
            drain_writeout(nloc - 2, 1)

        @pl.when((nloc >= 2) & jnp.logical_not(last_even))
        def _drain_d():
            drain_writeout(nloc - 2, 0)

    return k(td, ts, dst, src)


def _scatter_call(m0, m1, dst):
    ne = dst.shape[0]
    C = ne // KS

    @functools.partial(
        pl.kernel,
        out_type=(jax.ShapeDtypeStruct((N, 128), jnp.float32),
                  jax.ShapeDtypeStruct((N, 128), jnp.float32)),
        mesh=_sc_mesh(),
        scratch_types=[
            pltpu.VMEM((KS,), jnp.int32),
            pltpu.VMEM((KS, 128), jnp.float32),
            pltpu.VMEM((RW, 128), jnp.float32),
            pltpu.VMEM_SHARED((N, 128), jnp.float32),
        ],
    )
    def k(m0_h, m1_h, dst_h, agg0_h, agg1_h, idxb, mbuf, obuf, acc):
        cid = lax.axis_index("c")
        sid = lax.axis_index("s")

        def zrow(r, carry):
            def zlane(j, c2):
                obuf[r, pl.ds(j * 16, 16)] = jnp.zeros((16,), jnp.float32)
                return c2
            return lax.fori_loop(0, 128 // 16, zlane, carry)

        lax.fori_loop(0, RW, zrow, 0)
        nw = (CW - sid + NS - 1) // NS

        def zchunk(j, carry):
            t = sid + j * NS
            pltpu.sync_copy(obuf, acc.at[pl.ds(t * RW, RW)])
            return carry

        lax.fori_loop(0, nw, zchunk, 0)
        plsc.subcore_barrier()

        nloc = (C - sid + NS - 1) // NS

        def body(j, carry):
            c = sid + j * NS
            base = c * KS
            pltpu.sync_copy(dst_h.at[pl.ds(base, KS)], idxb)

            @pl.when(cid == 0)
            def _l0():
                pltpu.sync_copy(m0_h.at[pl.ds(base, KS)], mbuf)

            @pl.when(cid == 1)
            def _l1():
                pltpu.sync_copy(m1_h.at[pl.ds(base, KS)], mbuf)

            pltpu.sync_copy(mbuf, acc.at[idxb], add=True)
            return carry

        lax.fori_loop(0, nloc, body, 0)
        plsc.subcore_barrier()

        def ochunk(j, carry):
            r0 = (sid + j * NS) * RW
            pltpu.sync_copy(acc.at[pl.ds(r0, RW)], obuf)

            @pl.when(cid == 0)
            def _s0():
                pltpu.sync_copy(obuf, agg0_h.at[pl.ds(r0, RW)])

            @pl.when(cid == 1)
            def _s1():
                pltpu.sync_copy(obuf, agg1_h.at[pl.ds(r0, RW)])

            return carry

        lax.fori_loop(0, nw, ochunk, 0)

    return k(m0, m1, dst)


def kernel(x, edge_index, edge_attr, energies, batch, emb, We1, We2, Wsk,
           Wf, Ws, Wfe1, Wfe2, Wfc1, Wfc2, be1, be2, bfe1, bfe2, bfc1, bfc2):
    src = edge_index[0].astype(jnp.int32)
    dst = edge_index[1].astype(jnp.int32)
    x3 = x.astype(jnp.int32).reshape(N // BN, 1, BN)
    b3 = batch.astype(jnp.int32).reshape(N // BN, 1, BN)

    wd = []
    wsrc = []
    wedge = []
    for i in range(3):
        wd.append(jnp.concatenate([Wf[i, :D, :], Ws[i, :D, :]], axis=1))
        wsrc.append(jnp.concatenate([Wf[i, D:2 * D, :], Ws[i, D:2 * D, :]], axis=1))
        wedge.append(jnp.concatenate([Wf[i, 2 * D:, :], Ws[i, 2 * D:, :]], axis=1))

    be1r = be1.reshape(1, 128)
    be2r = be2.reshape(1, D)

    EH = E // 2
    dsth = (dst[:EH], dst[EH:])
    srch = (src[:EH], src[EH:])
    eah = (edge_attr[:EH], edge_attr[EH:])

    td, ts, s0, s1 = _node0_call(x3, emb, wd[0], wsrc[0], Wsk[0])
    aggs = None
    for i in range(3):
        mh = []
        aggs = []
        for h in range(2):
            g = _gatheradd_call(td, ts, dsth[h], srch[h])
            mh.append(_edge_call(eah[h], g, We1, be1r, We2, be2r, wedge[i]))
        for h in range(2):
            aggs.append(_scatter_call(mh[h][0], mh[h][1], dsth[h]))
        (a0a, a1a), (a0b, a1b) = aggs
        if i < 2:
            td, ts, s0n, s1n = _node12_call(
                a0a, a1a, a0b, a1b, s0, s1, wd[i + 1], wsrc[i + 1], Wsk[i + 1])
            s0, s1 = s0n, s1n

    (a0a, a1a), (a0b, a1b) = aggs
    sump, maxp, cnt = _pool_call(a0a, a1a, a0b, a1b, s0, s1, b3)
    out = _head_call(
        sump, maxp, cnt, energies, Wfe1, bfe1.reshape(1, D),
        Wfe2, bfe2.reshape(1, 128), Wfc1, bfc1.reshape(1, 1024),
        Wfc2, bfc2.reshape(1, 804))
    return out.reshape(G, 4, 201)


# speedup vs baseline: 2.4988x; 1.0890x over previous
import functools

import jax
import jax.numpy as jnp
from jax import lax
from jax.experimental import pallas as pl
from jax.experimental.pallas import tpu as pltpu
from jax.experimental.pallas import tpu_sc as plsc

N = 10000
E = 160000
G = 16
D = 256

NC = 2
NS = 16
NW = NC * NS

BN = 2000
BE = 2000
KG = 40
KS = 128
RW = 80
CW = N // RW


def _leaky(v):
    return jnp.where(v >= 0, v, 0.01 * v)


def _softplus(v):
    return jnp.maximum(v, 0.0) + jnp.log1p(jnp.exp(-jnp.abs(v)))


def _node0_body(x_ref, emb_ref, wd_ref, ws_ref, wsk_ref,
                td_ref, ts_ref, s0_ref, s1_ref):
    xb = x_ref[0, 0, :].reshape(BN, 1)
    oh = (xb == lax.broadcasted_iota(jnp.int32, (BN, 118), 1)).astype(jnp.float32)
    h = jnp.dot(oh, emb_ref[...], preferred_element_type=jnp.float32)
    td_ref[...] = jnp.dot(h, wd_ref[...], preferred_element_type=jnp.float32)
    ts_ref[...] = jnp.dot(h, ws_ref[...], preferred_element_type=jnp.float32)
    s = jnp.dot(h, wsk_ref[...], preferred_element_type=jnp.float32)
    s0_ref[...] = s[:, :128]
    s1_ref[...] = s[:, 128:]


def _node0_call(x3, emb, wd, ws, wsk):
    return pl.pallas_call(
        _node0_body,
        grid=(N // BN,),
        in_specs=[
            pl.BlockSpec((1, 1, BN), lambda i: (i, 0, 0)),
            pl.BlockSpec((118, D), lambda i: (0, 0)),
            pl.BlockSpec((D, 2 * D), lambda i: (0, 0)),
            pl.BlockSpec((D, 2 * D), lambda i: (0, 0)),
            pl.BlockSpec((D, D), lambda i: (0, 0)),
        ],
        out_specs=[
            pl.BlockSpec((BN, 2 * D), lambda i: (i, 0)),
            pl.BlockSpec((BN, 2 * D), lambda i: (i, 0)),
            pl.BlockSpec((BN, 128), lambda i: (i, 0)),
            pl.BlockSpec((BN, 128), lambda i: (i, 0)),
        ],
        out_shape=[
            jax.ShapeDtypeStruct((N, 2 * D), jnp.float32),
            jax.ShapeDtypeStruct((N, 2 * D), jnp.float32),
            jax.ShapeDtypeStruct((N, 128), jnp.float32),
            jax.ShapeDtypeStruct((N, 128), jnp.float32),
        ],
    )(x3, emb, wd, ws, wsk)


def _node12_body(a0a_ref, a1a_ref, a0b_ref, a1b_ref, p0_ref, p1_ref,
                 wd_ref, ws_ref, wsk_ref,
                 td_ref, ts_ref, s0_ref, s1_ref):
    h = jnp.concatenate(
        [a0a_ref[...] + a0b_ref[...] + p0_ref[...],
         a1a_ref[...] + a1b_ref[...] + p1_ref[...]], axis=1)
    td_ref[...] = jnp.dot(h, wd_ref[...], preferred_element_type=jnp.float32)
    ts_ref[...] = jnp.dot(h, ws_ref[...], preferred_element_type=jnp.float32)
    s = jnp.dot(h, wsk_ref[...], preferred_element_type=jnp.float32)
    s0_ref[...] = s[:, :128]
    s1_ref[...] = s[:, 128:]


def _node12_call(a0a, a1a, a0b, a1b, p0, p1, wd, ws, wsk):
    half = pl.BlockSpec((BN, 128), lambda i: (i, 0))
    return pl.pallas_call(
        _node12_body,
        grid=(N // BN,),
        in_specs=[
            half, half, half, half, half, half,
            pl.BlockSpec((D, 2 * D), lambda i: (0, 0)),
            pl.BlockSpec((D, 2 * D), lambda i: (0, 0)),
            pl.BlockSpec((D, D), lambda i: (0, 0)),
        ],
        out_specs=[
            pl.BlockSpec((BN, 2 * D), lambda i: (i, 0)),
            pl.BlockSpec((BN, 2 * D), lambda i: (i, 0)),
            pl.BlockSpec((BN, 128), lambda i: (i, 0)),
            pl.BlockSpec((BN, 128), lambda i: (i, 0)),
        ],
        out_shape=[
            jax.ShapeDtypeStruct((N, 2 * D), jnp.float32),
            jax.ShapeDtypeStruct((N, 2 * D), jnp.float32),
            jax.ShapeDtypeStruct((N, 128), jnp.float32),
            jax.ShapeDtypeStruct((N, 128), jnp.float32),
        ],
    )(a0a, a1a, a0b, a1b, p0, p1, wd, ws, wsk)


def _edge_body(ea_ref, g_ref, we1_ref, be1_ref, we2_ref, be2_ref,
               wedge_ref, m0_ref, m1_ref):
    e0 = jnp.dot(ea_ref[...], we1_ref[...],
                 preferred_element_type=jnp.float32) + be1_ref[...]
    e1 = jnp.dot(_leaky(e0), we2_ref[...],
                 preferred_element_type=jnp.float32) + be2_ref[...]
    pq = jnp.dot(e1, wedge_ref[...], preferred_element_type=jnp.float32)
    pq = pq + g_ref[...]
    p = pq[:, :D]
    q = pq[:, D:]
    m = (1.0 / (1.0 + jnp.exp(-p))) * _softplus(q)
    m0_ref[...] = m[:, :128]
    m1_ref[...] = m[:, 128:]


def _edge_call(edge_attr, g, we1, be1, we2, be2, wedge):
    ne = edge_attr.shape[0]
    return pl.pallas_call(
        _edge_body,
        grid=(ne // BE,),
        in_specs=[
            pl.BlockSpec((BE, 14), lambda i: (i, 0)),
            pl.BlockSpec((BE, 2 * D), lambda i: (i, 0)),
            pl.BlockSpec((14, 128), lambda i: (0, 0)),
            pl.BlockSpec((1, 128), lambda i: (0, 0)),
            pl.BlockSpec((128, D), lambda i: (0, 0)),
            pl.BlockSpec((1, D), lambda i: (0, 0)),
            pl.BlockSpec((D, 2 * D), lambda i: (0, 0)),
        ],
        out_specs=[
            pl.BlockSpec((BE, 128), lambda i: (i, 0)),
            pl.BlockSpec((BE, 128), lambda i: (i, 0)),
        ],
        out_shape=[
            jax.ShapeDtypeStruct((ne, 128), jnp.float32),
            jax.ShapeDtypeStruct((ne, 128), jnp.float32),
        ],
    )(edge_attr, g, we1, be1, we2, be2, wedge)


def _pool_body(a0a_ref, a1a_ref, a0b_ref, a1b_ref, p0_ref, p1_ref, b_ref,
               sum_ref, max_ref, cnt_ref):
    i = pl.program_id(0)

    @pl.when(i == 0)
    def _init():
        sum_ref[...] = jnp.zeros((G, D), jnp.float32)
        max_ref[...] = jnp.full((G, D), -jnp.inf, jnp.float32)
        cnt_ref[...] = jnp.zeros((G, 128), jnp.float32)

    h = jnp.concatenate(
        [a0a_ref[...] + a0b_ref[...] + p0_ref[...],
         a1a_ref[...] + a1b_ref[...] + p1_ref[...]], axis=1)
    bb = b_ref[0, 0, :].reshape(BN, 1)
    oh = (bb == lax.broadcasted_iota(jnp.int32, (BN, G), 1)).astype(jnp.float32)
    sum_ref[...] += lax.dot_general(
        oh, h, (((0,), (0,)), ((), ())), preferred_element_type=jnp.float32)
    cnt_ref[...] += jnp.broadcast_to(
        jnp.sum(oh, axis=0).reshape(G, 1), (G, 128))
    for g in range(G):
        sel = jnp.where(oh[:, g:g + 1] > 0, h, -jnp.inf)
        row = jnp.max(sel, axis=0).reshape(1, D)
        max_ref[g:g + 1, :] = jnp.maximum(max_ref[g:g + 1, :], row)


def _pool_call(a0a, a1a, a0b, a1b, p0, p1, b3):
    half = pl.BlockSpec((BN, 128), lambda i: (i, 0))
    return pl.pallas_call(
        _pool_body,
        grid=(N // BN,),
        in_specs=[
            half, half, half, half, half, half,
            pl.BlockSpec((1, 1, BN), lambda i: (i, 0, 0)),
        ],
        out_specs=[
            pl.BlockSpec((G, D), lambda i: (0, 0)),
            pl.BlockSpec((G, D), lambda i: (0, 0)),
            pl.BlockSpec((G, 128), lambda i: (0, 0)),
        ],
        out_shape=[
            jax.ShapeDtypeStruct((G, D), jnp.float32),
            jax.ShapeDtypeStruct((G, D), jnp.float32),
            jax.ShapeDtypeStruct((G, 128), jnp.float32),
        ],
    )(a0a, a1a, a0b, a1b, p0, p1, b3)


def _head_body(sum_ref, max_ref, cnt_ref, en_ref, wfe1_ref, bfe1_ref,
               wfe2_ref, bfe2_ref, wfc1_ref, bfc1_ref, wfc2_ref, bfc2_ref,
               out_ref):
    en = jnp.dot(en_ref[...], wfe1_ref[...],
                 preferred_element_type=jnp.float32) + bfe1_ref[...]
    en = jnp.dot(_leaky(en), wfe2_ref[...],
                 preferred_element_type=jnp.float32) + bfe2_ref[...]
    cnt = cnt_ref[...][:, 0:1]
    sump = sum_ref[...]
    meanp = sump / jnp.maximum(cnt, 1.0)
    crys = jnp.concatenate([meanp, max_ref[...], sump, en], axis=1)
    hid = jnp.dot(crys, wfc1_ref[...],
                  preferred_element_type=jnp.float32) + bfc1_ref[...]
    out_ref[...] = jnp.dot(_leaky(hid), wfc2_ref[...],
                           preferred_element_type=jnp.float32) + bfc2_ref[...]


def _head_call(sump, maxp, cnt, energies, wfe1, bfe1, wfe2, bfe2,
               wfc1, bfc1, wfc2, bfc2):
    full = lambda a: pl.BlockSpec(a.shape, lambda: tuple(0 for _ in a.shape))
    args = (sump, maxp, cnt, energies, wfe1, bfe1, wfe2, bfe2,
            wfc1, bfc1, wfc2, bfc2)
    return pl.pallas_call(
        _head_body,
        in_specs=[full(a) for a in args],
        out_specs=pl.BlockSpec((G, 804), lambda: (0, 0)),
        out_shape=jax.ShapeDtypeStruct((G, 804), jnp.float32),
    )(*args)


def _sc_mesh():
    return plsc.VectorSubcoreMesh(
        core_axis_name="c", subcore_axis_name="s",
        num_cores=NC, num_subcores=NS)


def _gatheradd_call(td, ts, dst, src):
    ne = dst.shape[0]
    C = ne // KG

    @functools.partial(
        pl.kernel,
        out_type=jax.ShapeDtypeStruct((ne, 2 * D), jnp.float32),
        mesh=_sc_mesh(),
        scratch_types=[
            pltpu.VMEM((KG,), jnp.int32),
            pltpu.VMEM((KG,), jnp.int32),
            pltpu.VMEM((KG,), jnp.int32),
            pltpu.VMEM((KG,), jnp.int32),
            pltpu.VMEM((KG, 2 * D), jnp.float32),
            pltpu.VMEM((KG, 2 * D), jnp.float32),
            pltpu.VMEM((KG, 2 * D), jnp.float32),
            pltpu.VMEM((KG, 2 * D), jnp.float32),
            pltpu.SemaphoreType.DMA,
            pltpu.SemaphoreType.DMA,
            pltpu.SemaphoreType.DMA,
            pltpu.SemaphoreType.DMA,
        ],
    )
    def k(td_h, ts_h, dst_h, src_h, g_h, i_d0, i_s0, i_d1, i_s1,
          bd0, bs0, bd1, bs1, gsem0, gsem1, wsem0, wsem1):
        wid = lax.axis_index("s") * NC + lax.axis_index("c")
        nloc = (C - wid + NW - 1) // NW
        idx = ((i_d0, i_s0), (i_d1, i_s1))
        bufs = ((bd0, bs0), (bd1, bs1))
        gsems = (gsem0, gsem1)
        wsems = (wsem0, wsem1)

        def base_of(j):
            return (wid + j * NW) * KG

        def stage_and_fire(j, slot):
            base = base_of(j)
            pltpu.sync_copy(dst_h.at[pl.ds(base, KG)], idx[slot][0])
            pltpu.sync_copy(src_h.at[pl.ds(base, KG)], idx[slot][1])
            pltpu.async_copy(td_h.at[idx[slot][0]], bufs[slot][0], gsems[slot])
            pltpu.async_copy(ts_h.at[idx[slot][1]], bufs[slot][1], gsems[slot])

        def wait_gathers(slot):
            pltpu.make_async_copy(
                td_h.at[idx[slot][0]], bufs[slot][0], gsems[slot]).wait()
            pltpu.make_async_copy(
                ts_h.at[idx[slot][1]], bufs[slot][1], gsems[slot]).wait()

        def drain_writeout(j, slot):
            pltpu.make_async_copy(
                bufs[slot][0], g_h.at[pl.ds(base_of(j), KG)],
                wsems[slot]).wait()

        stage_and_fire(0, 0)

        def pair(j2, carry):
            for b in range(2):
                j = j2 * 2 + b
                slot = b
                other = 1 - b

                @pl.when(j < nloc)
                def _step():
                    wait_gathers(slot)

                    @pl.when(j + 1 < nloc)
                    def _fire_next():
                        @pl.when(j >= 1)
                        def _drain_prev():
                            drain_writeout(j - 1, other)

                        stage_and_fire(j + 1, other)

                    bd, bs = bufs[slot]

                    @plsc.parallel_loop(0, KG)
                    def _add(r):
                        for t in range(2 * D // 16):
                            sl = pl.ds(t * 16, 16)
                            bd[r, sl] = bd[r, sl] + bs[r, sl]

                    pltpu.async_copy(
                        bufs[slot][0], g_h.at[pl.ds(base_of(j), KG)],
                        wsems[slot])
            return carry

        lax.fori_loop(0, (nloc + 1) // 2, pair, 0)

        last_even = (nloc - 1) % 2 == 0

        @pl.when((nloc >= 1) & last_even)
        def _drain_a():
            drain_writeout(nloc - 1, 0)

        @pl.when((nloc >= 1) & jnp.logical_not(last_even))
        def _drain_b():
            drain_writeout(nloc - 1, 1)

        @pl.when((nloc >= 2) & last_even)
        def _drain_c():
            drain_writeout(nloc - 2, 1)

        @pl.when((nloc >= 2) & jnp.logical_not(last_even))
        def _drain_d():
            drain_writeout(nloc - 2, 0)

    return k(td, ts, dst, src)


def _scatter_call(m0, m1, dst):
    ne = dst.shape[0]
    C = ne // KS

    @functools.partial(
        pl.kernel,
        out_type=(jax.ShapeDtypeStruct((N, 128), jnp.float32),
                  jax.ShapeDtypeStruct((N, 128), jnp.float32)),
        mesh=_sc_mesh(),
        scratch_types=[
            pltpu.VMEM((KS,), jnp.int32),
            pltpu.VMEM((KS,), jnp.int32),
            pltpu.VMEM((KS, 128), jnp.float32),
            pltpu.VMEM((KS, 128), jnp.float32),
            pltpu.VMEM((RW, 128), jnp.float32),
            pltpu.VMEM_SHARED((N, 128), jnp.float32),
            pltpu.SemaphoreType.DMA,
            pltpu.SemaphoreType.DMA,
            pltpu.SemaphoreType.DMA,
            pltpu.SemaphoreType.DMA,
        ],
    )
    def k(m0_h, m1_h, dst_h, agg0_h, agg1_h, idx0, idx1, mb0, mb1, obuf, acc,
          lsem0, lsem1, ssem0, ssem1):
        cid = lax.axis_index("c")
        sid = lax.axis_index("s")
        idx = (idx0, idx1)
        mbuf = (mb0, mb1)
        lsems = (lsem0, lsem1)
        ssems = (ssem0, ssem1)

        def zrow(r, carry):
            def zlane(j, c2):
                obuf[r, pl.ds(j * 16, 16)] = jnp.zeros((16,), jnp.float32)
                return c2
            return lax.fori_loop(0, 128 // 16, zlane, carry)

        lax.fori_loop(0, RW, zrow, 0)
        nw = (CW - sid + NS - 1) // NS

        def zchunk(j, carry):
            t = sid + j * NS
            pltpu.sync_copy(obuf, acc.at[pl.ds(t * RW, RW)])
            return carry

        lax.fori_loop(0, nw, zchunk, 0)
        plsc.subcore_barrier()

        nloc = (C - sid + NS - 1) // NS

        def base_of(j):
            return (sid + j * NS) * KS

        def fire_loads(j, slot):
            base = base_of(j)
            pltpu.async_copy(dst_h.at[pl.ds(base, KS)], idx[slot], lsems[slot])

            @pl.when(cid == 0)
            def _l0():
                pltpu.async_copy(m0_h.at[pl.ds(base, KS)], mbuf[slot],
                                 lsems[slot])

            @pl.when(cid == 1)
            def _l1():
                pltpu.async_copy(m1_h.at[pl.ds(base, KS)], mbuf[slot],
                                 lsems[slot])

        def wait_loads(j, slot):
            base = base_of(j)
            pltpu.make_async_copy(
                dst_h.at[pl.ds(base, KS)], idx[slot], lsems[slot]).wait()
            pltpu.make_async_copy(
                m0_h.at[pl.ds(base, KS)], mbuf[slot], lsems[slot]).wait()

        def drain_scatter(slot):
            pltpu.make_async_copy(mbuf[slot], acc.at[idx[slot]],
                                  ssems[slot]).wait()

        fire_loads(0, 0)

        def pair(j2, carry):
            for b in range(2):
                j = j2 * 2 + b
                slot = b
                other = 1 - b

                @pl.when(j < nloc)
                def _step():
                    wait_loads(j, slot)

                    @pl.when(j + 1 < nloc)
                    def _fire_next():
                        @pl.when(j >= 1)
                        def _drain_prev():
                            drain_scatter(other)

                        fire_loads(j + 1, other)

                    pltpu.async_copy(mbuf[slot], acc.at[idx[slot]],
                                     ssems[slot], add=True)
            return carry

        lax.fori_loop(0, (nloc + 1) // 2, pair, 0)

        last_even = (nloc - 1) % 2 == 0

        @pl.when((nloc >= 1) & last_even)
        def _drain_a():
            drain_scatter(0)

        @pl.when((nloc >= 1) & jnp.logical_not(last_even))
        def _drain_b():
            drain_scatter(1)

        @pl.when((nloc >= 2) & last_even)
        def _drain_c():
            drain_scatter(1)

        @pl.when((nloc >= 2) & jnp.logical_not(last_even))
        def _drain_d():
            drain_scatter(0)

        plsc.subcore_barrier()

        def ochunk(j, carry):
            r0 = (sid + j * NS) * RW
            pltpu.sync_copy(acc.at[pl.ds(r0, RW)], obuf)

            @pl.when(cid == 0)
            def _s0():
                pltpu.sync_copy(obuf, agg0_h.at[pl.ds(r0, RW)])

            @pl.when(cid == 1)
            def _s1():
                pltpu.sync_copy(obuf, agg1_h.at[pl.ds(r0, RW)])

            return carry

        lax.fori_loop(0, nw, ochunk, 0)

    return k(m0, m1, dst)


def kernel(x, edge_index, edge_attr, energies, batch, emb, We1, We2, Wsk,
           Wf, Ws, Wfe1, Wfe2, Wfc1, Wfc2, be1, be2, bfe1, bfe2, bfc1, bfc2):
    src = edge_index[0].astype(jnp.int32)
    dst = edge_index[1].astype(jnp.int32)
    x3 = x.astype(jnp.int32).reshape(N // BN, 1, BN)
    b3 = batch.astype(jnp.int32).reshape(N // BN, 1, BN)

    wd = []
    wsrc = []
    wedge = []
    for i in range(3):
        wd.append(jnp.concatenate([Wf[i, :D, :], Ws[i, :D, :]], axis=1))
        wsrc.append(jnp.concatenate([Wf[i, D:2 * D, :], Ws[i, D:2 * D, :]], axis=1))
        wedge.append(jnp.concatenate([Wf[i, 2 * D:, :], Ws[i, 2 * D:, :]], axis=1))

    be1r = be1.reshape(1, 128)
    be2r = be2.reshape(1, D)

    EH = E // 2
    dsth = (dst[:EH], dst[EH:])
    srch = (src[:EH], src[EH:])
    eah = (edge_attr[:EH], edge_attr[EH:])

    td, ts, s0, s1 = _node0_call(x3, emb, wd[0], wsrc[0], Wsk[0])
    aggs = None
    for i in range(3):
        mh = []
        aggs = []
        for h in range(2):
            g = _gatheradd_call(td, ts, dsth[h], srch[h])
            mh.append(_edge_call(eah[h], g, We1, be1r, We2, be2r, wedge[i]))
        for h in range(2):
            aggs.append(_scatter_call(mh[h][0], mh[h][1], dsth[h]))
        (a0a, a1a), (a0b, a1b) = aggs
        if i < 2:
            td, ts, s0n, s1n = _node12_call(
                a0a, a1a, a0b, a1b, s0, s1, wd[i + 1], wsrc[i + 1], Wsk[i + 1])
            s0, s1 = s0n, s1n

    (a0a, a1a), (a0b, a1b) = aggs
    sump, maxp, cnt = _pool_call(a0a, a1a, a0b, a1b, s0, s1, b3)
    out = _head_call(
        sump, maxp, cnt, energies, Wfe1, bfe1.reshape(1, D),
        Wfe2, bfe2.reshape(1, 128), Wfc1, bfc1.reshape(1, 1024),
        Wfc2, bfc2.reshape(1, 804))
    return out.reshape(G, 4, 201)


# speedup vs baseline: 3.5762x; 1.4312x over previous
import functools

import jax
import jax.numpy as jnp
from jax import lax
from jax.experimental import pallas as pl
from jax.experimental.pallas import tpu as pltpu
from jax.experimental.pallas import tpu_sc as plsc

N = 10000
E = 160000
G = 16
D = 256

NC = 2
NS = 16
NW = NC * NS

BN = 2000
BE = 2000
KG = 64
KS = 128
RW = 80
CW = N // RW


def _leaky(v):
    return jnp.where(v >= 0, v, 0.01 * v)


def _pack2(a, b):
    ra = lax.bitcast_convert_type(
        a.astype(jnp.bfloat16).astype(jnp.float32), jnp.int32)
    rb = lax.bitcast_convert_type(
        b.astype(jnp.bfloat16).astype(jnp.float32), jnp.int32)
    return ra | ((rb >> 16) & 0xFFFF)


def _unpack_hi(w):
    return lax.bitcast_convert_type(w & jnp.int32(-65536), jnp.float32)


def _unpack_lo(w):
    return lax.bitcast_convert_type(w << 16, jnp.float32)


def _softplus(v):
    return jnp.maximum(v, 0.0) + jnp.log1p(jnp.exp(-jnp.abs(v)))


def _node0_body(x_ref, emb_ref, wd_ref, ws_ref, wsk_ref,
                td_ref, ts_ref, s0_ref, s1_ref):
    xb = x_ref[0, 0, :].reshape(BN, 1)
    oh = (xb == lax.broadcasted_iota(jnp.int32, (BN, 118), 1)).astype(jnp.float32)
    h = jnp.dot(oh, emb_ref[...], preferred_element_type=jnp.float32)
    tdf = jnp.dot(h, wd_ref[...], preferred_element_type=jnp.float32)
    td_ref[...] = _pack2(tdf[:, :D], tdf[:, D:])
    tsf = jnp.dot(h, ws_ref[...], preferred_element_type=jnp.float32)
    ts_ref[...] = _pack2(tsf[:, :D], tsf[:, D:])
    s = jnp.dot(h, wsk_ref[...], preferred_element_type=jnp.float32)
    s0_ref[...] = s[:, :128]
    s1_ref[...] = s[:, 128:]


def _node0_call(x3, emb, wd, ws, wsk):
    return pl.pallas_call(
        _node0_body,
        grid=(N // BN,),
        in_specs=[
            pl.BlockSpec((1, 1, BN), lambda i: (i, 0, 0)),
            pl.BlockSpec((118, D), lambda i: (0, 0)),
            pl.BlockSpec((D, 2 * D), lambda i: (0, 0)),
            pl.BlockSpec((D, 2 * D), lambda i: (0, 0)),
            pl.BlockSpec((D, D), lambda i: (0, 0)),
        ],
        out_specs=[
            pl.BlockSpec((BN, D), lambda i: (i, 0)),
            pl.BlockSpec((BN, D), lambda i: (i, 0)),
            pl.BlockSpec((BN, 128), lambda i: (i, 0)),
            pl.BlockSpec((BN, 128), lambda i: (i, 0)),
        ],
        out_shape=[
            jax.ShapeDtypeStruct((N, D), jnp.int32),
            jax.ShapeDtypeStruct((N, D), jnp.int32),
            jax.ShapeDtypeStruct((N, 128), jnp.float32),
            jax.ShapeDtypeStruct((N, 128), jnp.float32),
        ],
    )(x3, emb, wd, ws, wsk)


def _node12_body(a0a_ref, a1a_ref, a0b_ref, a1b_ref, p0_ref, p1_ref,
                 wd_ref, ws_ref, wsk_ref,
                 td_ref, ts_ref, s0_ref, s1_ref):
    h = jnp.concatenate(
        [a0a_ref[...] + a0b_ref[...] + p0_ref[...],
         a1a_ref[...] + a1b_ref[...] + p1_ref[...]], axis=1)
    tdf = jnp.dot(h, wd_ref[...], preferred_element_type=jnp.float32)
    td_ref[...] = _pack2(tdf[:, :D], tdf[:, D:])
    tsf = jnp.dot(h, ws_ref[...], preferred_element_type=jnp.float32)
    ts_ref[...] = _pack2(tsf[:, :D], tsf[:, D:])
    s = jnp.dot(h, wsk_ref[...], preferred_element_type=jnp.float32)
    s0_ref[...] = s[:, :128]
    s1_ref[...] = s[:, 128:]


def _node12_call(a0a, a1a, a0b, a1b, p0, p1, wd, ws, wsk):
    half = pl.BlockSpec((BN, 128), lambda i: (i, 0))
    return pl.pallas_call(
        _node12_body,
        grid=(N // BN,),
        in_specs=[
            half, half, half, half, half, half,
            pl.BlockSpec((D, 2 * D), lambda i: (0, 0)),
            pl.BlockSpec((D, 2 * D), lambda i: (0, 0)),
            pl.BlockSpec((D, D), lambda i: (0, 0)),
        ],
        out_specs=[
            pl.BlockSpec((BN, D), lambda i: (i, 0)),
            pl.BlockSpec((BN, D), lambda i: (i, 0)),
            pl.BlockSpec((BN, 128), lambda i: (i, 0)),
            pl.BlockSpec((BN, 128), lambda i: (i, 0)),
        ],
        out_shape=[
            jax.ShapeDtypeStruct((N, D), jnp.int32),
            jax.ShapeDtypeStruct((N, D), jnp.int32),
            jax.ShapeDtypeStruct((N, 128), jnp.float32),
            jax.ShapeDtypeStruct((N, 128), jnp.float32),
        ],
    )(a0a, a1a, a0b, a1b, p0, p1, wd, ws, wsk)


def _edge_body(ea_ref, g_ref, we1_ref, be1_ref, we2_ref, be2_ref,
               wedge_ref, m0_ref, m1_ref):
    e0 = jnp.dot(ea_ref[...], we1_ref[...],
                 preferred_element_type=jnp.float32) + be1_ref[...]
    e1 = jnp.dot(_leaky(e0), we2_ref[...],
                 preferred_element_type=jnp.float32) + be2_ref[...]
    pq = jnp.dot(e1, wedge_ref[...], preferred_element_type=jnp.float32)
    gw = g_ref[...]
    p = pq[:, :D] + _unpack_hi(gw)
    q = pq[:, D:] + _unpack_lo(gw)
    m = (1.0 / (1.0 + jnp.exp(-p))) * _softplus(q)
    m0_ref[...] = m[:, :128]
    m1_ref[...] = m[:, 128:]


def _edge_call(edge_attr, g, we1, be1, we2, be2, wedge):
    ne = edge_attr.shape[0]
    return pl.pallas_call(
        _edge_body,
        grid=(ne // BE,),
        in_specs=[
            pl.BlockSpec((BE, 14), lambda i: (i, 0)),
            pl.BlockSpec((BE, D), lambda i: (i, 0)),
            pl.BlockSpec((14, 128), lambda i: (0, 0)),
            pl.BlockSpec((1, 128), lambda i: (0, 0)),
            pl.BlockSpec((128, D), lambda i: (0, 0)),
            pl.BlockSpec((1, D), lambda i: (0, 0)),
            pl.BlockSpec((D, 2 * D), lambda i: (0, 0)),
        ],
        out_specs=[
            pl.BlockSpec((BE, 128), lambda i: (i, 0)),
            pl.BlockSpec((BE, 128), lambda i: (i, 0)),
        ],
        out_shape=[
            jax.ShapeDtypeStruct((ne, 128), jnp.float32),
            jax.ShapeDtypeStruct((ne, 128), jnp.float32),
        ],
    )(edge_attr, g, we1, be1, we2, be2, wedge)


def _pool_body(a0a_ref, a1a_ref, a0b_ref, a1b_ref, p0_ref, p1_ref, b_ref,
               sum_ref, max_ref, cnt_ref):
    i = pl.program_id(0)

    @pl.when(i == 0)
    def _init():
        sum_ref[...] = jnp.zeros((G, D), jnp.float32)
        max_ref[...] = jnp.full((G, D), -jnp.inf, jnp.float32)
        cnt_ref[...] = jnp.zeros((G, 128), jnp.float32)

    h = jnp.concatenate(
        [a0a_ref[...] + a0b_ref[...] + p0_ref[...],
         a1a_ref[...] + a1b_ref[...] + p1_ref[...]], axis=1)
    bb = b_ref[0, 0, :].reshape(BN, 1)
    oh = (bb == lax.broadcasted_iota(jnp.int32, (BN, G), 1)).astype(jnp.float32)
    sum_ref[...] += lax.dot_general(
        oh, h, (((0,), (0,)), ((), ())), preferred_element_type=jnp.float32)
    cnt_ref[...] += jnp.broadcast_to(
        jnp.sum(oh, axis=0).reshape(G, 1), (G, 128))
    for g in range(G):
        sel = jnp.where(oh[:, g:g + 1] > 0, h, -jnp.inf)
        row = jnp.max(sel, axis=0).reshape(1, D)
        max_ref[g:g + 1, :] = jnp.maximum(max_ref[g:g + 1, :], row)


def _pool_call(a0a, a1a, a0b, a1b, p0, p1, b3):
    half = pl.BlockSpec((BN, 128), lambda i: (i, 0))
    return pl.pallas_call(
        _pool_body,
        grid=(N // BN,),
        in_specs=[
            half, half, half, half, half, half,
            pl.BlockSpec((1, 1, BN), lambda i: (i, 0, 0)),
        ],
        out_specs=[
            pl.BlockSpec((G, D), lambda i: (0, 0)),
            pl.BlockSpec((G, D), lambda i: (0, 0)),
            pl.BlockSpec((G, 128), lambda i: (0, 0)),
        ],
        out_shape=[
            jax.ShapeDtypeStruct((G, D), jnp.float32),
            jax.ShapeDtypeStruct((G, D), jnp.float32),
            jax.ShapeDtypeStruct((G, 128), jnp.float32),
        ],
    )(a0a, a1a, a0b, a1b, p0, p1, b3)


def _head_body(sum_ref, max_ref, cnt_ref, en_ref, wfe1_ref, bfe1_ref,
               wfe2_ref, bfe2_ref, wfc1_ref, bfc1_ref, wfc2_ref, bfc2_ref,
               out_ref):
    en = jnp.dot(en_ref[...], wfe1_ref[...],
                 preferred_element_type=jnp.float32) + bfe1_ref[...]
    en = jnp.dot(_leaky(en), wfe2_ref[...],
                 preferred_element_type=jnp.float32) + bfe2_ref[...]
    cnt = cnt_ref[...][:, 0:1]
    sump = sum_ref[...]
    meanp = sump / jnp.maximum(cnt, 1.0)
    crys = jnp.concatenate([meanp, max_ref[...], sump, en], axis=1)
    hid = jnp.dot(crys, wfc1_ref[...],
                  preferred_element_type=jnp.float32) + bfc1_ref[...]
    out_ref[...] = jnp.dot(_leaky(hid), wfc2_ref[...],
                           preferred_element_type=jnp.float32) + bfc2_ref[...]


def _head_call(sump, maxp, cnt, energies, wfe1, bfe1, wfe2, bfe2,
               wfc1, bfc1, wfc2, bfc2):
    full = lambda a: pl.BlockSpec(a.shape, lambda: tuple(0 for _ in a.shape))
    args = (sump, maxp, cnt, energies, wfe1, bfe1, wfe2, bfe2,
            wfc1, bfc1, wfc2, bfc2)
    return pl.pallas_call(
        _head_body,
        in_specs=[full(a) for a in args],
        out_specs=pl.BlockSpec((G, 804), lambda: (0, 0)),
        out_shape=jax.ShapeDtypeStruct((G, 804), jnp.float32),
    )(*args)


def _sc_mesh():
    return plsc.VectorSubcoreMesh(
        core_axis_name="c", subcore_axis_name="s",
        num_cores=NC, num_subcores=NS)


def _gatheradd_call(td, ts, dst, src):
    ne = dst.shape[0]
    C = ne // KG

    @functools.partial(
        pl.kernel,
        out_type=jax.ShapeDtypeStruct((ne, D), jnp.int32),
        mesh=_sc_mesh(),
        compiler_params=pltpu.CompilerParams(needs_layout_passes=False),
        scratch_types=[
            pltpu.VMEM((KG,), jnp.int32),
            pltpu.VMEM((KG,), jnp.int32),
            pltpu.VMEM((KG,), jnp.int32),
            pltpu.VMEM((KG,), jnp.int32),
            pltpu.VMEM((KG, D), jnp.int32),
            pltpu.VMEM((KG, D), jnp.int32),
            pltpu.VMEM((KG, D), jnp.int32),
            pltpu.VMEM((KG, D), jnp.int32),
            pltpu.SemaphoreType.DMA,
            pltpu.SemaphoreType.DMA,
            pltpu.SemaphoreType.DMA,
            pltpu.SemaphoreType.DMA,
        ],
    )
    def k(td_h, ts_h, dst_h, src_h, g_h, i_d0, i_s0, i_d1, i_s1,
          bd0, bs0, bd1, bs1, gsem0, gsem1, wsem0, wsem1):
        wid = lax.axis_index("s") * NC + lax.axis_index("c")
        nloc = (C - wid + NW - 1) // NW
        idx = ((i_d0, i_s0), (i_d1, i_s1))
        bufs = ((bd0, bs0), (bd1, bs1))
        gsems = (gsem0, gsem1)
        wsems = (wsem0, wsem1)

        def base_of(j):
            return (wid + j * NW) * KG

        def stage_and_fire(j, slot):
            base = base_of(j)
            pltpu.sync_copy(dst_h.at[pl.ds(base, KG)], idx[slot][0])
            pltpu.sync_copy(src_h.at[pl.ds(base, KG)], idx[slot][1])
            pltpu.async_copy(td_h.at[idx[slot][0]], bufs[slot][0], gsems[slot])
            pltpu.async_copy(ts_h.at[idx[slot][1]], bufs[slot][1], gsems[slot])

        def wait_gathers(slot):
            pltpu.make_async_copy(
                td_h.at[idx[slot][0]], bufs[slot][0], gsems[slot]).wait()
            pltpu.make_async_copy(
                ts_h.at[idx[slot][1]], bufs[slot][1], gsems[slot]).wait()

        def drain_writeout(j, slot):
            pltpu.make_async_copy(
                bufs[slot][0], g_h.at[pl.ds(base_of(j), KG)],
                wsems[slot]).wait()

        stage_and_fire(0, 0)

        def pair(j2, carry):
            for b in range(2):
                j = j2 * 2 + b
                slot = b
                other = 1 - b

                @pl.when(j < nloc)
                def _step():
                    wait_gathers(slot)

                    @pl.when(j + 1 < nloc)
                    def _fire_next():
                        @pl.when(j >= 1)
                        def _drain_prev():
                            drain_writeout(j - 1, other)

                        stage_and_fire(j + 1, other)

                    bd, bs = bufs[slot]

                    @plsc.parallel_loop(0, KG)
                    def _add(r):
                        for t in range(D // 16):
                            sl = pl.ds(t * 16, 16)
                            a = plsc.bitcast(bd[r, sl], jnp.bfloat16)
                            b = plsc.bitcast(bs[r, sl], jnp.bfloat16)
                            bd[r, sl] = plsc.bitcast(a + b, jnp.int32)

                    pltpu.async_copy(
                        bufs[slot][0], g_h.at[pl.ds(base_of(j), KG)],
                        wsems[slot])
            return carry

        lax.fori_loop(0, (nloc + 1) // 2, pair, 0)

        last_even = (nloc - 1) % 2 == 0

        @pl.when((nloc >= 1) & last_even)
        def _drain_a():
            drain_writeout(nloc - 1, 0)

        @pl.when((nloc >= 1) & jnp.logical_not(last_even))
        def _drain_b():
            drain_writeout(nloc - 1, 1)

        @pl.when((nloc >= 2) & last_even)
        def _drain_c():
            drain_writeout(nloc - 2, 1)

        @pl.when((nloc >= 2) & jnp.logical_not(last_even))
        def _drain_d():
            drain_writeout(nloc - 2, 0)

    return k(td, ts, dst, src)


def _scatter_call(m0, m1, dst):
    ne = dst.shape[0]
    C = ne // KS

    @functools.partial(
        pl.kernel,
        out_type=(jax.ShapeDtypeStruct((N, 128), jnp.float32),
                  jax.ShapeDtypeStruct((N, 128), jnp.float32)),
        mesh=_sc_mesh(),
        scratch_types=[
            pltpu.VMEM((KS,), jnp.int32),
            pltpu.VMEM((KS,), jnp.int32),
            pltpu.VMEM((KS, 128), jnp.float32),
            pltpu.VMEM((KS, 128), jnp.float32),
            pltpu.VMEM((RW, 128), jnp.float32),
            pltpu.VMEM_SHARED((N, 128), jnp.float32),
            pltpu.SemaphoreType.DMA,
            pltpu.SemaphoreType.DMA,
            pltpu.SemaphoreType.DMA,
            pltpu.SemaphoreType.DMA,
        ],
    )
    def k(m0_h, m1_h, dst_h, agg0_h, agg1_h, idx0, idx1, mb0, mb1, obuf, acc,
          lsem0, lsem1, ssem0, ssem1):
        cid = lax.axis_index("c")
        sid = lax.axis_index("s")
        idx = (idx0, idx1)
        mbuf = (mb0, mb1)
        lsems = (lsem0, lsem1)
        ssems = (ssem0, ssem1)

        def zrow(r, carry):
            def zlane(j, c2):
                obuf[r, pl.ds(j * 16, 16)] = jnp.zeros((16,), jnp.float32)
                return c2
            return lax.fori_loop(0, 128 // 16, zlane, carry)

        lax.fori_loop(0, RW, zrow, 0)
        nw = (CW - sid + NS - 1) // NS

        def zchunk(j, carry):
            t = sid + j * NS
            pltpu.sync_copy(obuf, acc.at[pl.ds(t * RW, RW)])
            return carry

        lax.fori_loop(0, nw, zchunk, 0)
        plsc.subcore_barrier()

        nloc = (C - sid + NS - 1) // NS

        def base_of(j):
            return (sid + j * NS) * KS

        def fire_loads(j, slot):
            base = base_of(j)
            pltpu.async_copy(dst_h.at[pl.ds(base, KS)], idx[slot], lsems[slot])

            @pl.when(cid == 0)
            def _l0():
                pltpu.async_copy(m0_h.at[pl.ds(base, KS)], mbuf[slot],
                                 lsems[slot])

            @pl.when(cid == 1)
            def _l1():
                pltpu.async_copy(m1_h.at[pl.ds(base, KS)], mbuf[slot],
                                 lsems[slot])

        def wait_loads(j, slot):
            base = base_of(j)
            pltpu.make_async_copy(
                dst_h.at[pl.ds(base, KS)], idx[slot], lsems[slot]).wait()
            pltpu.make_async_copy(
                m0_h.at[pl.ds(base, KS)], mbuf[slot], lsems[slot]).wait()

        def drain_scatter(slot):
            pltpu.make_async_copy(mbuf[slot], acc.at[idx[slot]],
                                  ssems[slot]).wait()

        fire_loads(0, 0)

        def pair(j2, carry):
            for b in range(2):
                j = j2 * 2 + b
                slot = b
                other = 1 - b

                @pl.when(j < nloc)
                def _step():
                    wait_loads(j, slot)

                    @pl.when(j + 1 < nloc)
                    def _fire_next():
                        @pl.when(j >= 1)
                        def _drain_prev():
                            drain_scatter(other)

                        fire_loads(j + 1, other)

                    pltpu.async_copy(mbuf[slot], acc.at[idx[slot]],
                                     ssems[slot], add=True)
            return carry

        lax.fori_loop(0, (nloc + 1) // 2, pair, 0)

        last_even = (nloc - 1) % 2 == 0

        @pl.when((nloc >= 1) & last_even)
        def _drain_a():
            drain_scatter(0)

        @pl.when((nloc >= 1) & jnp.logical_not(last_even))
        def _drain_b():
            drain_scatter(1)

        @pl.when((nloc >= 2) & last_even)
        def _drain_c():
            drain_scatter(1)

        @pl.when((nloc >= 2) & jnp.logical_not(last_even))
        def _drain_d():
            drain_scatter(0)

        plsc.subcore_barrier()

        def ochunk(j, carry):
            r0 = (sid + j * NS) * RW
            pltpu.sync_copy(acc.at[pl.ds(r0, RW)], obuf)

            @pl.when(cid == 0)
            def _s0():
                pltpu.sync_copy(obuf, agg0_h.at[pl.ds(r0, RW)])

            @pl.when(cid == 1)
            def _s1():
                pltpu.sync_copy(obuf, agg1_h.at[pl.ds(r0, RW)])

            return carry

        lax.fori_loop(0, nw, ochunk, 0)

    return k(m0, m1, dst)


def kernel(x, edge_index, edge_attr, energies, batch, emb, We1, We2, Wsk,
           Wf, Ws, Wfe1, Wfe2, Wfc1, Wfc2, be1, be2, bfe1, bfe2, bfc1, bfc2):
    src = edge_index[0].astype(jnp.int32)
    dst = edge_index[1].astype(jnp.int32)
    x3 = x.astype(jnp.int32).reshape(N // BN, 1, BN)
    b3 = batch.astype(jnp.int32).reshape(N // BN, 1, BN)

    wd = []
    wsrc = []
    wedge = []
    for i in range(3):
        wd.append(jnp.concatenate([Wf[i, :D, :], Ws[i, :D, :]], axis=1))
        wsrc.append(jnp.concatenate([Wf[i, D:2 * D, :], Ws[i, D:2 * D, :]], axis=1))
        wedge.append(jnp.concatenate([Wf[i, 2 * D:, :], Ws[i, 2 * D:, :]], axis=1))

    be1r = be1.reshape(1, 128)
    be2r = be2.reshape(1, D)

    EH = E // 2
    dsth = (dst[:EH], dst[EH:])
    srch = (src[:EH], src[EH:])
    eah = (edge_attr[:EH], edge_attr[EH:])

    td, ts, s0, s1 = _node0_call(x3, emb, wd[0], wsrc[0], Wsk[0])
    aggs = None
    for i in range(3):
        mh = []
        aggs = []
        for h in range(2):
            g = _gatheradd_call(td, ts, dsth[h], srch[h])
            mh.append(_edge_call(eah[h], g, We1, be1r, We2, be2r, wedge[i]))
        for h in range(2):
            aggs.append(_scatter_call(mh[h][0], mh[h][1], dsth[h]))
        (a0a, a1a), (a0b, a1b) = aggs
        if i < 2:
            td, ts, s0n, s1n = _node12_call(
                a0a, a1a, a0b, a1b, s0, s1, wd[i + 1], wsrc[i + 1], Wsk[i + 1])
            s0, s1 = s0n, s1n

    (a0a, a1a), (a0b, a1b) = aggs
    sump, maxp, cnt = _pool_call(a0a, a1a, a0b, a1b, s0, s1, b3)
    out = _head_call(
        sump, maxp, cnt, energies, Wfe1, bfe1.reshape(1, D),
        Wfe2, bfe2.reshape(1, 128), Wfc1, bfc1.reshape(1, 1024),
        Wfc2, bfc2.reshape(1, 804))
    return out.reshape(G, 4, 201)


# speedup vs baseline: 3.9202x; 1.0962x over previous
import functools

import jax
import jax.numpy as jnp
from jax import lax
from jax.experimental import pallas as pl
from jax.experimental.pallas import tpu as pltpu
from jax.experimental.pallas import tpu_sc as plsc

N = 10000
E = 160000
G = 16
D = 256

NC = 2
NS = 16
NW = NC * NS

BN = 2000
BE = 2000
KG = 64
KS = 128
RW = 80
CW = N // RW


def _leaky(v):
    return jnp.where(v >= 0, v, 0.01 * v)


def _pack2(a, b):
    ra = lax.bitcast_convert_type(
        a.astype(jnp.bfloat16).astype(jnp.float32), jnp.int32)
    rb = lax.bitcast_convert_type(
        b.astype(jnp.bfloat16).astype(jnp.float32), jnp.int32)
    return ra | ((rb >> 16) & 0xFFFF)


def _unpack_hi(w):
    return lax.bitcast_convert_type(w & jnp.int32(-65536), jnp.float32)


def _unpack_lo(w):
    return lax.bitcast_convert_type(w << 16, jnp.float32)


def _softplus(v):
    return jnp.maximum(v, 0.0) + jnp.log1p(jnp.exp(-jnp.abs(v)))


def _node0_body(x_ref, emb_ref, wd_ref, ws_ref, wsk_ref,
                td_ref, ts_ref, s0_ref, s1_ref):
    xb = x_ref[0, 0, :].reshape(BN, 1)
    oh = (xb == lax.broadcasted_iota(jnp.int32, (BN, 118), 1)).astype(jnp.float32)
    h = jnp.dot(oh, emb_ref[...], preferred_element_type=jnp.float32)
    tdf = jnp.dot(h, wd_ref[...], preferred_element_type=jnp.float32)
    td_ref[...] = _pack2(tdf[:, :D], tdf[:, D:])
    tsf = jnp.dot(h, ws_ref[...], preferred_element_type=jnp.float32)
    ts_ref[...] = _pack2(tsf[:, :D], tsf[:, D:])
    s = jnp.dot(h, wsk_ref[...], preferred_element_type=jnp.float32)
    s0_ref[...] = s[:, :128]
    s1_ref[...] = s[:, 128:]


def _node0_call(x3, emb, wd, ws, wsk):
    return pl.pallas_call(
        _node0_body,
        grid=(N // BN,),
        in_specs=[
            pl.BlockSpec((1, 1, BN), lambda i: (i, 0, 0)),
            pl.BlockSpec((118, D), lambda i: (0, 0)),
            pl.BlockSpec((D, 2 * D), lambda i: (0, 0)),
            pl.BlockSpec((D, 2 * D), lambda i: (0, 0)),
            pl.BlockSpec((D, D), lambda i: (0, 0)),
        ],
        out_specs=[
            pl.BlockSpec((BN, D), lambda i: (i, 0)),
            pl.BlockSpec((BN, D), lambda i: (i, 0)),
            pl.BlockSpec((BN, 128), lambda i: (i, 0)),
            pl.BlockSpec((BN, 128), lambda i: (i, 0)),
        ],
        out_shape=[
            jax.ShapeDtypeStruct((N, D), jnp.int32),
            jax.ShapeDtypeStruct((N, D), jnp.int32),
            jax.ShapeDtypeStruct((N, 128), jnp.float32),
            jax.ShapeDtypeStruct((N, 128), jnp.float32),
        ],
    )(x3, emb, wd, ws, wsk)


def _node12_body(a0a_ref, a1a_ref, a0b_ref, a1b_ref, p0_ref, p1_ref,
                 wd_ref, ws_ref, wsk_ref,
                 td_ref, ts_ref, s0_ref, s1_ref):
    h = jnp.concatenate(
        [a0a_ref[...] + a0b_ref[...] + p0_ref[...],
         a1a_ref[...] + a1b_ref[...] + p1_ref[...]], axis=1)
    tdf = jnp.dot(h, wd_ref[...], preferred_element_type=jnp.float32)
    td_ref[...] = _pack2(tdf[:, :D], tdf[:, D:])
    tsf = jnp.dot(h, ws_ref[...], preferred_element_type=jnp.float32)
    ts_ref[...] = _pack2(tsf[:, :D], tsf[:, D:])
    s = jnp.dot(h, wsk_ref[...], preferred_element_type=jnp.float32)
    s0_ref[...] = s[:, :128]
    s1_ref[...] = s[:, 128:]


def _node12_call(a0a, a1a, a0b, a1b, p0, p1, wd, ws, wsk):
    half = pl.BlockSpec((BN, 128), lambda i: (i, 0))
    return pl.pallas_call(
        _node12_body,
        grid=(N // BN,),
        in_specs=[
            half, half, half, half, half, half,
            pl.BlockSpec((D, 2 * D), lambda i: (0, 0)),
            pl.BlockSpec((D, 2 * D), lambda i: (0, 0)),
            pl.BlockSpec((D, D), lambda i: (0, 0)),
        ],
        out_specs=[
            pl.BlockSpec((BN, D), lambda i: (i, 0)),
            pl.BlockSpec((BN, D), lambda i: (i, 0)),
            pl.BlockSpec((BN, 128), lambda i: (i, 0)),
            pl.BlockSpec((BN, 128), lambda i: (i, 0)),
        ],
        out_shape=[
            jax.ShapeDtypeStruct((N, D), jnp.int32),
            jax.ShapeDtypeStruct((N, D), jnp.int32),
            jax.ShapeDtypeStruct((N, 128), jnp.float32),
            jax.ShapeDtypeStruct((N, 128), jnp.float32),
        ],
    )(a0a, a1a, a0b, a1b, p0, p1, wd, ws, wsk)


def _edge_body(ea_ref, g_ref, we1_ref, be1_ref, we2_ref, be2_ref,
               wedge_ref, m0_ref, m1_ref):
    e0 = jnp.dot(ea_ref[...], we1_ref[...],
                 preferred_element_type=jnp.float32) + be1_ref[...]
    e1 = jnp.dot(_leaky(e0), we2_ref[...],
                 preferred_element_type=jnp.float32) + be2_ref[...]
    pq = jnp.dot(e1, wedge_ref[...], preferred_element_type=jnp.float32)
    gw = g_ref[...]
    p = pq[:, :D] + _unpack_hi(gw)
    q = pq[:, D:] + _unpack_lo(gw)
    m = (1.0 / (1.0 + jnp.exp(-p))) * _softplus(q)
    m0_ref[...] = m[:, :128]
    m1_ref[...] = m[:, 128:]


def _edge_call(edge_attr, g, we1, be1, we2, be2, wedge):
    ne = edge_attr.shape[0]
    return pl.pallas_call(
        _edge_body,
        grid=(ne // BE,),
        in_specs=[
            pl.BlockSpec((BE, 14), lambda i: (i, 0)),
            pl.BlockSpec((BE, D), lambda i: (i, 0)),
            pl.BlockSpec((14, 128), lambda i: (0, 0)),
            pl.BlockSpec((1, 128), lambda i: (0, 0)),
            pl.BlockSpec((128, D), lambda i: (0, 0)),
            pl.BlockSpec((1, D), lambda i: (0, 0)),
            pl.BlockSpec((D, 2 * D), lambda i: (0, 0)),
        ],
        out_specs=[
            pl.BlockSpec((BE, 128), lambda i: (i, 0)),
            pl.BlockSpec((BE, 128), lambda i: (i, 0)),
        ],
        out_shape=[
            jax.ShapeDtypeStruct((ne, 128), jnp.float32),
            jax.ShapeDtypeStruct((ne, 128), jnp.float32),
        ],
    )(edge_attr, g, we1, be1, we2, be2, wedge)


def _pool_body(a0a_ref, a1a_ref, a0b_ref, a1b_ref, p0_ref, p1_ref, b_ref,
               sum_ref, max_ref, cnt_ref):
    i = pl.program_id(0)

    @pl.when(i == 0)
    def _init():
        sum_ref[...] = jnp.zeros((G, D), jnp.float32)
        max_ref[...] = jnp.full((G, D), -jnp.inf, jnp.float32)
        cnt_ref[...] = jnp.zeros((G, 128), jnp.float32)

    h = jnp.concatenate(
        [a0a_ref[...] + a0b_ref[...] + p0_ref[...],
         a1a_ref[...] + a1b_ref[...] + p1_ref[...]], axis=1)
    bb = b_ref[0, 0, :].reshape(BN, 1)
    oh = (bb == lax.broadcasted_iota(jnp.int32, (BN, G), 1)).astype(jnp.float32)
    sum_ref[...] += lax.dot_general(
        oh, h, (((0,), (0,)), ((), ())), preferred_element_type=jnp.float32)
    cnt_ref[...] += jnp.broadcast_to(
        jnp.sum(oh, axis=0).reshape(G, 1), (G, 128))
    for g in range(G):
        sel = jnp.where(oh[:, g:g + 1] > 0, h, -jnp.inf)
        row = jnp.max(sel, axis=0).reshape(1, D)
        max_ref[g:g + 1, :] = jnp.maximum(max_ref[g:g + 1, :], row)


def _pool_call(a0a, a1a, a0b, a1b, p0, p1, b3):
    half = pl.BlockSpec((BN, 128), lambda i: (i, 0))
    return pl.pallas_call(
        _pool_body,
        grid=(N // BN,),
        in_specs=[
            half, half, half, half, half, half,
            pl.BlockSpec((1, 1, BN), lambda i: (i, 0, 0)),
        ],
        out_specs=[
            pl.BlockSpec((G, D), lambda i: (0, 0)),
            pl.BlockSpec((G, D), lambda i: (0, 0)),
            pl.BlockSpec((G, 128), lambda i: (0, 0)),
        ],
        out_shape=[
            jax.ShapeDtypeStruct((G, D), jnp.float32),
            jax.ShapeDtypeStruct((G, D), jnp.float32),
            jax.ShapeDtypeStruct((G, 128), jnp.float32),
        ],
    )(a0a, a1a, a0b, a1b, p0, p1, b3)


def _head_body(sum_ref, max_ref, cnt_ref, en_ref, wfe1_ref, bfe1_ref,
               wfe2_ref, bfe2_ref, wfc1_ref, bfc1_ref, wfc2_ref, bfc2_ref,
               out_ref):
    en = jnp.dot(en_ref[...], wfe1_ref[...],
                 preferred_element_type=jnp.float32) + bfe1_ref[...]
    en = jnp.dot(_leaky(en), wfe2_ref[...],
                 preferred_element_type=jnp.float32) + bfe2_ref[...]
    cnt = cnt_ref[...][:, 0:1]
    sump = sum_ref[...]
    meanp = sump / jnp.maximum(cnt, 1.0)
    crys = jnp.concatenate([meanp, max_ref[...], sump, en], axis=1)
    hid = jnp.dot(crys, wfc1_ref[...],
                  preferred_element_type=jnp.float32) + bfc1_ref[...]
    out_ref[...] = jnp.dot(_leaky(hid), wfc2_ref[...],
                           preferred_element_type=jnp.float32) + bfc2_ref[...]


def _head_call(sump, maxp, cnt, energies, wfe1, bfe1, wfe2, bfe2,
               wfc1, bfc1, wfc2, bfc2):
    full = lambda a: pl.BlockSpec(a.shape, lambda: tuple(0 for _ in a.shape))
    args = (sump, maxp, cnt, energies, wfe1, bfe1, wfe2, bfe2,
            wfc1, bfc1, wfc2, bfc2)
    return pl.pallas_call(
        _head_body,
        in_specs=[full(a) for a in args],
        out_specs=pl.BlockSpec((G, 804), lambda: (0, 0)),
        out_shape=jax.ShapeDtypeStruct((G, 804), jnp.float32),
    )(*args)


def _sc_mesh():
    return plsc.VectorSubcoreMesh(
        core_axis_name="c", subcore_axis_name="s",
        num_cores=NC, num_subcores=NS)


def _gatheradd_call(td, ts, dst, src):
    ne = dst.shape[0]
    C = ne // KG
    cpw = C // NW
    rem = C - cpw * NW
    nidx = (cpw + 1) * KG

    @functools.partial(
        pl.kernel,
        out_type=jax.ShapeDtypeStruct((ne, D), jnp.int32),
        mesh=_sc_mesh(),
        compiler_params=pltpu.CompilerParams(needs_layout_passes=False),
        scratch_types=[
            pltpu.VMEM((nidx,), jnp.int32),
            pltpu.VMEM((nidx,), jnp.int32),
            pltpu.VMEM((KG, D), jnp.int32),
            pltpu.VMEM((KG, D), jnp.int32),
            pltpu.VMEM((KG, D), jnp.int32),
            pltpu.VMEM((KG, D), jnp.int32),
            pltpu.SemaphoreType.DMA,
            pltpu.SemaphoreType.DMA,
            pltpu.SemaphoreType.DMA,
            pltpu.SemaphoreType.DMA,
        ],
    )
    def k(td_h, ts_h, dst_h, src_h, g_h, idx_d, idx_s,
          bd0, bs0, bd1, bs1, gsem0, gsem1, wsem0, wsem1):
        wid = lax.axis_index("s") * NC + lax.axis_index("c")
        nloc = jnp.where(wid < rem, cpw + 1, cpw)
        bufs = ((bd0, bs0), (bd1, bs1))
        gsems = (gsem0, gsem1)
        wsems = (wsem0, wsem1)

        first = wid * cpw * KG
        pltpu.sync_copy(dst_h.at[pl.ds(first, cpw * KG)],
                        idx_d.at[pl.ds(0, cpw * KG)])
        pltpu.sync_copy(src_h.at[pl.ds(first, cpw * KG)],
                        idx_s.at[pl.ds(0, cpw * KG)])

        @pl.when(wid < rem)
        def _extra():
            eb = (cpw * NW + wid) * KG
            pltpu.sync_copy(dst_h.at[pl.ds(eb, KG)],
                            idx_d.at[pl.ds(cpw * KG, KG)])
            pltpu.sync_copy(src_h.at[pl.ds(eb, KG)],
                            idx_s.at[pl.ds(cpw * KG, KG)])

        def base_of(j):
            c = jnp.where(j < cpw, wid * cpw + j, cpw * NW + wid)
            return c * KG

        def stage_and_fire(j, slot):
            sl = pl.ds(j * KG, KG)
            pltpu.async_copy(td_h.at[idx_d.at[sl]], bufs[slot][0], gsems[slot])
            pltpu.async_copy(ts_h.at[idx_s.at[sl]], bufs[slot][1], gsems[slot])

        def wait_gathers(slot):
            z = pl.ds(0, KG)
            pltpu.make_async_copy(
                td_h.at[idx_d.at[z]], bufs[slot][0], gsems[slot]).wait()
            pltpu.make_async_copy(
                ts_h.at[idx_s.at[z]], bufs[slot][1], gsems[slot]).wait()

        def drain_writeout(j, slot):
            pltpu.make_async_copy(
                bufs[slot][0], g_h.at[pl.ds(base_of(j), KG)],
                wsems[slot]).wait()

        stage_and_fire(0, 0)

        def pair(j2, carry):
            for b in range(2):
                j = j2 * 2 + b
                slot = b
                other = 1 - b

                @pl.when(j < nloc)
                def _step():
                    wait_gathers(slot)

                    @pl.when(j + 1 < nloc)
                    def _fire_next():
                        @pl.when(j >= 1)
                        def _drain_prev():
                            drain_writeout(j - 1, other)

                        stage_and_fire(j + 1, other)

                    bd, bs = bufs[slot]

                    @plsc.parallel_loop(0, KG)
                    def _add(r):
                        for t in range(D // 16):
                            sl = pl.ds(t * 16, 16)
                            a = plsc.bitcast(bd[r, sl], jnp.bfloat16)
                            b = plsc.bitcast(bs[r, sl], jnp.bfloat16)
                            bd[r, sl] = plsc.bitcast(a + b, jnp.int32)

                    pltpu.async_copy(
                        bufs[slot][0], g_h.at[pl.ds(base_of(j), KG)],
                        wsems[slot])
            return carry

        lax.fori_loop(0, (nloc + 1) // 2, pair, 0)

        last_even = (nloc - 1) % 2 == 0

        @pl.when((nloc >= 1) & last_even)
        def _drain_a():
            drain_writeout(nloc - 1, 0)

        @pl.when((nloc >= 1) & jnp.logical_not(last_even))
        def _drain_b():
            drain_writeout(nloc - 1, 1)

        @pl.when((nloc >= 2) & last_even)
        def _drain_c():
            drain_writeout(nloc - 2, 1)

        @pl.when((nloc >= 2) & jnp.logical_not(last_even))
        def _drain_d():
            drain_writeout(nloc - 2, 0)

    return k(td, ts, dst, src)


def _scatter_call(m0, m1, dst):
    ne = dst.shape[0]
    C = ne // KS

    @functools.partial(
        pl.kernel,
        out_type=(jax.ShapeDtypeStruct((N, 128), jnp.float32),
                  jax.ShapeDtypeStruct((N, 128), jnp.float32)),
        mesh=_sc_mesh(),
        scratch_types=[
            pltpu.VMEM((KS,), jnp.int32),
            pltpu.VMEM((KS,), jnp.int32),
            pltpu.VMEM((KS, 128), jnp.float32),
            pltpu.VMEM((KS, 128), jnp.float32),
            pltpu.VMEM((RW, 128), jnp.float32),
            pltpu.VMEM_SHARED((N, 128), jnp.float32),
            pltpu.SemaphoreType.DMA,
            pltpu.SemaphoreType.DMA,
            pltpu.SemaphoreType.DMA,
            pltpu.SemaphoreType.DMA,
        ],
    )
    def k(m0_h, m1_h, dst_h, agg0_h, agg1_h, idx0, idx1, mb0, mb1, obuf, acc,
          lsem0, lsem1, ssem0, ssem1):
        cid = lax.axis_index("c")
        sid = lax.axis_index("s")
        idx = (idx0, idx1)
        mbuf = (mb0, mb1)
        lsems = (lsem0, lsem1)
        ssems = (ssem0, ssem1)

        def zrow(r, carry):
            def zlane(j, c2):
                obuf[r, pl.ds(j * 16, 16)] = jnp.zeros((16,), jnp.float32)
                return c2
            return lax.fori_loop(0, 128 // 16, zlane, carry)

        lax.fori_loop(0, RW, zrow, 0)
        nw = (CW - sid + NS - 1) // NS

        def zchunk(j, carry):
            t = sid + j * NS
            pltpu.sync_copy(obuf, acc.at[pl.ds(t * RW, RW)])
            return carry

        lax.fori_loop(0, nw, zchunk, 0)
        plsc.subcore_barrier()

        nloc = (C - sid + NS - 1) // NS

        def base_of(j):
            return (sid + j * NS) * KS

        def fire_loads(j, slot):
            base = base_of(j)
            pltpu.async_copy(dst_h.at[pl.ds(base, KS)], idx[slot], lsems[slot])

            @pl.when(cid == 0)
            def _l0():
                pltpu.async_copy(m0_h.at[pl.ds(base, KS)], mbuf[slot],
                                 lsems[slot])

            @pl.when(cid == 1)
            def _l1():
                pltpu.async_copy(m1_h.at[pl.ds(base, KS)], mbuf[slot],
                                 lsems[slot])

        def wait_loads(j, slot):
            base = base_of(j)
            pltpu.make_async_copy(
                dst_h.at[pl.ds(base, KS)], idx[slot], lsems[slot]).wait()
            pltpu.make_async_copy(
                m0_h.at[pl.ds(base, KS)], mbuf[slot], lsems[slot]).wait()

        def drain_scatter(slot):
            pltpu.make_async_copy(mbuf[slot], acc.at[idx[slot]],
                                  ssems[slot]).wait()

        fire_loads(0, 0)

        def pair(j2, carry):
            for b in range(2):
                j = j2 * 2 + b
                slot = b
                other = 1 - b

                @pl.when(j < nloc)
                def _step():
                    wait_loads(j, slot)

                    @pl.when(j + 1 < nloc)
                    def _fire_next():
                        @pl.when(j >= 1)
                        def _drain_prev():
                            drain_scatter(other)

                        fire_loads(j + 1, other)

                    pltpu.async_copy(mbuf[slot], acc.at[idx[slot]],
                                     ssems[slot], add=True)
            return carry

        lax.fori_loop(0, (nloc + 1) // 2, pair, 0)

        last_even = (nloc - 1) % 2 == 0

        @pl.when((nloc >= 1) & last_even)
        def _drain_a():
            drain_scatter(0)

        @pl.when((nloc >= 1) & jnp.logical_not(last_even))
        def _drain_b():
            drain_scatter(1)

        @pl.when((nloc >= 2) & last_even)
        def _drain_c():
            drain_scatter(1)

        @pl.when((nloc >= 2) & jnp.logical_not(last_even))
        def _drain_d():
            drain_scatter(0)

        plsc.subcore_barrier()

        def ochunk(j, carry):
            r0 = (sid + j * NS) * RW
            pltpu.sync_copy(acc.at[pl.ds(r0, RW)], obuf)

            @pl.when(cid == 0)
            def _s0():
                pltpu.sync_copy(obuf, agg0_h.at[pl.ds(r0, RW)])

            @pl.when(cid == 1)
            def _s1():
                pltpu.sync_copy(obuf, agg1_h.at[pl.ds(r0, RW)])

            return carry

        lax.fori_loop(0, nw, ochunk, 0)

    return k(m0, m1, dst)


def kernel(x, edge_index, edge_attr, energies, batch, emb, We1, We2, Wsk,
           Wf, Ws, Wfe1, Wfe2, Wfc1, Wfc2, be1, be2, bfe1, bfe2, bfc1, bfc2):
    src = edge_index[0].astype(jnp.int32)
    dst = edge_index[1].astype(jnp.int32)
    x3 = x.astype(jnp.int32).reshape(N // BN, 1, BN)
    b3 = batch.astype(jnp.int32).reshape(N // BN, 1, BN)

    wd = []
    wsrc = []
    wedge = []
    for i in range(3):
        wd.append(jnp.concatenate([Wf[i, :D, :], Ws[i, :D, :]], axis=1))
        wsrc.append(jnp.concatenate([Wf[i, D:2 * D, :], Ws[i, D:2 * D, :]], axis=1))
        wedge.append(jnp.concatenate([Wf[i, 2 * D:, :], Ws[i, 2 * D:, :]], axis=1))

    be1r = be1.reshape(1, 128)
    be2r = be2.reshape(1, D)

    EH = E // 2
    dsth = (dst[:EH], dst[EH:])
    srch = (src[:EH], src[EH:])
    eah = (edge_attr[:EH], edge_attr[EH:])

    td, ts, s0, s1 = _node0_call(x3, emb, wd[0], wsrc[0], Wsk[0])
    aggs = None
    for i in range(3):
        mh = []
        aggs = []
        for h in range(2):
            g = _gatheradd_call(td, ts, dsth[h], srch[h])
            mh.append(_edge_call(eah[h], g, We1, be1r, We2, be2r, wedge[i]))
        for h in range(2):
            aggs.append(_scatter_call(mh[h][0], mh[h][1], dsth[h]))
        (a0a, a1a), (a0b, a1b) = aggs
        if i < 2:
            td, ts, s0n, s1n = _node12_call(
                a0a, a1a, a0b, a1b, s0, s1, wd[i + 1], wsrc[i + 1], Wsk[i + 1])
            s0, s1 = s0n, s1n

    (a0a, a1a), (a0b, a1b) = aggs
    sump, maxp, cnt = _pool_call(a0a, a1a, a0b, a1b, s0, s1, b3)
    out = _head_call(
        sump, maxp, cnt, energies, Wfe1, bfe1.reshape(1, D),
        Wfe2, bfe2.reshape(1, 128), Wfc1, bfc1.reshape(1, 1024),
        Wfc2, bfc2.reshape(1, 804))
    return out.reshape(G, 4, 201)


# speedup vs baseline: 3.9347x; 1.0037x over previous
import functools

import jax
import jax.numpy as jnp
from jax import lax
from jax.experimental import pallas as pl
from jax.experimental.pallas import tpu as pltpu
from jax.experimental.pallas import tpu_sc as plsc

N = 10000
E = 160000
G = 16
D = 256

NC = 2
NS = 16
NW = NC * NS

BN = 2000
BE = 2000
KG = 80
KS = 128
RW = 80
CW = N // RW


def _leaky(v):
    return jnp.where(v >= 0, v, 0.01 * v)


def _pack2(a, b):
    ra = lax.bitcast_convert_type(
        a.astype(jnp.bfloat16).astype(jnp.float32), jnp.int32)
    rb = lax.bitcast_convert_type(
        b.astype(jnp.bfloat16).astype(jnp.float32), jnp.int32)
    return ra | ((rb >> 16) & 0xFFFF)


def _unpack_hi(w):
    return lax.bitcast_convert_type(w & jnp.int32(-65536), jnp.float32)


def _unpack_lo(w):
    return lax.bitcast_convert_type(w << 16, jnp.float32)


def _softplus(v):
    return jnp.maximum(v, 0.0) + jnp.log1p(jnp.exp(-jnp.abs(v)))


def _node0_body(x_ref, emb_ref, wd_ref, ws_ref, wsk_ref,
                td_ref, ts_ref, s0_ref, s1_ref):
    xb = x_ref[0, 0, :].reshape(BN, 1)
    oh = (xb == lax.broadcasted_iota(jnp.int32, (BN, 118), 1)).astype(jnp.float32)
    h = jnp.dot(oh, emb_ref[...], preferred_element_type=jnp.float32)
    tdf = jnp.dot(h, wd_ref[...], preferred_element_type=jnp.float32)
    td_ref[...] = _pack2(tdf[:, :D], tdf[:, D:])
    tsf = jnp.dot(h, ws_ref[...], preferred_element_type=jnp.float32)
    ts_ref[...] = _pack2(tsf[:, :D], tsf[:, D:])
    s = jnp.dot(h, wsk_ref[...], preferred_element_type=jnp.float32)
    s0_ref[...] = s[:, :128]
    s1_ref[...] = s[:, 128:]


def _node0_call(x3, emb, wd, ws, wsk):
    return pl.pallas_call(
        _node0_body,
        grid=(N // BN,),
        in_specs=[
            pl.BlockSpec((1, 1, BN), lambda i: (i, 0, 0)),
            pl.BlockSpec((118, D), lambda i: (0, 0)),
            pl.BlockSpec((D, 2 * D), lambda i: (0, 0)),
            pl.BlockSpec((D, 2 * D), lambda i: (0, 0)),
            pl.BlockSpec((D, D), lambda i: (0, 0)),
        ],
        out_specs=[
            pl.BlockSpec((BN, D), lambda i: (i, 0)),
            pl.BlockSpec((BN, D), lambda i: (i, 0)),
            pl.BlockSpec((BN, 128), lambda i: (i, 0)),
            pl.BlockSpec((BN, 128), lambda i: (i, 0)),
        ],
        out_shape=[
            jax.ShapeDtypeStruct((N, D), jnp.int32),
            jax.ShapeDtypeStruct((N, D), jnp.int32),
            jax.ShapeDtypeStruct((N, 128), jnp.float32),
            jax.ShapeDtypeStruct((N, 128), jnp.float32),
        ],
    )(x3, emb, wd, ws, wsk)


def _node12_body(a0a_ref, a1a_ref, a0b_ref, a1b_ref, p0_ref, p1_ref,
                 wd_ref, ws_ref, wsk_ref,
                 td_ref, ts_ref, s0_ref, s1_ref):
    h = jnp.concatenate(
        [a0a_ref[...] + a0b_ref[...] + p0_ref[...],
         a1a_ref[...] + a1b_ref[...] + p1_ref[...]], axis=1)
    tdf = jnp.dot(h, wd_ref[...], preferred_element_type=jnp.float32)
    td_ref[...] = _pack2(tdf[:, :D], tdf[:, D:])
    tsf = jnp.dot(h, ws_ref[...], preferred_element_type=jnp.float32)
    ts_ref[...] = _pack2(tsf[:, :D], tsf[:, D:])
    s = jnp.dot(h, wsk_ref[...], preferred_element_type=jnp.float32)
    s0_ref[...] = s[:, :128]
    s1_ref[...] = s[:, 128:]


def _node12_call(a0a, a1a, a0b, a1b, p0, p1, wd, ws, wsk):
    half = pl.BlockSpec((BN, 128), lambda i: (i, 0))
    return pl.pallas_call(
        _node12_body,
        grid=(N // BN,),
        in_specs=[
            half, half, half, half, half, half,
            pl.BlockSpec((D, 2 * D), lambda i: (0, 0)),
            pl.BlockSpec((D, 2 * D), lambda i: (0, 0)),
            pl.BlockSpec((D, D), lambda i: (0, 0)),
        ],
        out_specs=[
            pl.BlockSpec((BN, D), lambda i: (i, 0)),
            pl.BlockSpec((BN, D), lambda i: (i, 0)),
            pl.BlockSpec((BN, 128), lambda i: (i, 0)),
            pl.BlockSpec((BN, 128), lambda i: (i, 0)),
        ],
        out_shape=[
            jax.ShapeDtypeStruct((N, D), jnp.int32),
            jax.ShapeDtypeStruct((N, D), jnp.int32),
            jax.ShapeDtypeStruct((N, 128), jnp.float32),
            jax.ShapeDtypeStruct((N, 128), jnp.float32),
        ],
    )(a0a, a1a, a0b, a1b, p0, p1, wd, ws, wsk)


def _edge_body(ea_ref, g_ref, we1_ref, be1_ref, we2_ref, be2_ref,
               wedge_ref, m0_ref, m1_ref):
    e0 = jnp.dot(ea_ref[...], we1_ref[...],
                 preferred_element_type=jnp.float32) + be1_ref[...]
    e1 = jnp.dot(_leaky(e0), we2_ref[...],
                 preferred_element_type=jnp.float32) + be2_ref[...]
    pq = jnp.dot(e1, wedge_ref[...], preferred_element_type=jnp.float32)
    gw = g_ref[...]
    p = pq[:, :D] + _unpack_hi(gw)
    q = pq[:, D:] + _unpack_lo(gw)
    m = (1.0 / (1.0 + jnp.exp(-p))) * _softplus(q)
    m0_ref[...] = m[:, :128]
    m1_ref[...] = m[:, 128:]


def _edge_call(edge_attr, g, we1, be1, we2, be2, wedge):
    ne = edge_attr.shape[0]
    return pl.pallas_call(
        _edge_body,
        grid=(ne // BE,),
        in_specs=[
            pl.BlockSpec((BE, 14), lambda i: (i, 0)),
            pl.BlockSpec((BE, D), lambda i: (i, 0)),
            pl.BlockSpec((14, 128), lambda i: (0, 0)),
            pl.BlockSpec((1, 128), lambda i: (0, 0)),
            pl.BlockSpec((128, D), lambda i: (0, 0)),
            pl.BlockSpec((1, D), lambda i: (0, 0)),
            pl.BlockSpec((D, 2 * D), lambda i: (0, 0)),
        ],
        out_specs=[
            pl.BlockSpec((BE, 128), lambda i: (i, 0)),
            pl.BlockSpec((BE, 128), lambda i: (i, 0)),
        ],
        out_shape=[
            jax.ShapeDtypeStruct((ne, 128), jnp.float32),
            jax.ShapeDtypeStruct((ne, 128), jnp.float32),
        ],
    )(edge_attr, g, we1, be1, we2, be2, wedge)


def _pool_body(a0a_ref, a1a_ref, a0b_ref, a1b_ref, p0_ref, p1_ref, b_ref,
               sum_ref, max_ref, cnt_ref):
    i = pl.program_id(0)

    @pl.when(i == 0)
    def _init():
        sum_ref[...] = jnp.zeros((G, D), jnp.float32)
        max_ref[...] = jnp.full((G, D), -jnp.inf, jnp.float32)
        cnt_ref[...] = jnp.zeros((G, 128), jnp.float32)

    h = jnp.concatenate(
        [a0a_ref[...] + a0b_ref[...] + p0_ref[...],
         a1a_ref[...] + a1b_ref[...] + p1_ref[...]], axis=1)
    bb = b_ref[0, 0, :].reshape(BN, 1)
    oh = (bb == lax.broadcasted_iota(jnp.int32, (BN, G), 1)).astype(jnp.float32)
    sum_ref[...] += lax.dot_general(
        oh, h, (((0,), (0,)), ((), ())), preferred_element_type=jnp.float32)
    cnt_ref[...] += jnp.broadcast_to(
        jnp.sum(oh, axis=0).reshape(G, 1), (G, 128))
    for g in range(G):
        sel = jnp.where(oh[:, g:g + 1] > 0, h, -jnp.inf)
        row = jnp.max(sel, axis=0).reshape(1, D)
        max_ref[g:g + 1, :] = jnp.maximum(max_ref[g:g + 1, :], row)


def _pool_call(a0a, a1a, a0b, a1b, p0, p1, b3):
    half = pl.BlockSpec((BN, 128), lambda i: (i, 0))
    return pl.pallas_call(
        _pool_body,
        grid=(N // BN,),
        in_specs=[
            half, half, half, half, half, half,
            pl.BlockSpec((1, 1, BN), lambda i: (i, 0, 0)),
        ],
        out_specs=[
            pl.BlockSpec((G, D), lambda i: (0, 0)),
            pl.BlockSpec((G, D), lambda i: (0, 0)),
            pl.BlockSpec((G, 128), lambda i: (0, 0)),
        ],
        out_shape=[
            jax.ShapeDtypeStruct((G, D), jnp.float32),
            jax.ShapeDtypeStruct((G, D), jnp.float32),
            jax.ShapeDtypeStruct((G, 128), jnp.float32),
        ],
    )(a0a, a1a, a0b, a1b, p0, p1, b3)


def _head_body(sum_ref, max_ref, cnt_ref, en_ref, wfe1_ref, bfe1_ref,
               wfe2_ref, bfe2_ref, wfc1_ref, bfc1_ref, wfc2_ref, bfc2_ref,
               out_ref):
    en = jnp.dot(en_ref[...], wfe1_ref[...],
                 preferred_element_type=jnp.float32) + bfe1_ref[...]
    en = jnp.dot(_leaky(en), wfe2_ref[...],
                 preferred_element_type=jnp.float32) + bfe2_ref[...]
    cnt = cnt_ref[...][:, 0:1]
    sump = sum_ref[...]
    meanp = sump / jnp.maximum(cnt, 1.0)
    crys = jnp.concatenate([meanp, max_ref[...], sump, en], axis=1)
    hid = jnp.dot(crys, wfc1_ref[...],
                  preferred_element_type=jnp.float32) + bfc1_ref[...]
    out_ref[...] = jnp.dot(_leaky(hid), wfc2_ref[...],
                           preferred_element_type=jnp.float32) + bfc2_ref[...]


def _head_call(sump, maxp, cnt, energies, wfe1, bfe1, wfe2, bfe2,
               wfc1, bfc1, wfc2, bfc2):
    full = lambda a: pl.BlockSpec(a.shape, lambda: tuple(0 for _ in a.shape))
    args = (sump, maxp, cnt, energies, wfe1, bfe1, wfe2, bfe2,
            wfc1, bfc1, wfc2, bfc2)
    return pl.pallas_call(
        _head_body,
        in_specs=[full(a) for a in args],
        out_specs=pl.BlockSpec((G, 804), lambda: (0, 0)),
        out_shape=jax.ShapeDtypeStruct((G, 804), jnp.float32),
    )(*args)


def _sc_mesh():
    return plsc.VectorSubcoreMesh(
        core_axis_name="c", subcore_axis_name="s",
        num_cores=NC, num_subcores=NS)


def _gatheradd_call(td, ts, dst, src):
    ne = dst.shape[0]
    C = ne // KG
    cpw = C // NW
    rem = C - cpw * NW
    nidx = (cpw + 1) * KG

    @functools.partial(
        pl.kernel,
        out_type=jax.ShapeDtypeStruct((ne, D), jnp.int32),
        mesh=_sc_mesh(),
        compiler_params=pltpu.CompilerParams(needs_layout_passes=False),
        scratch_types=[
            pltpu.VMEM((nidx,), jnp.int32),
            pltpu.VMEM((nidx,), jnp.int32),
            pltpu.VMEM((KG, D), jnp.int32),
            pltpu.VMEM((KG, D), jnp.int32),
            pltpu.VMEM((KG, D), jnp.int32),
            pltpu.VMEM((KG, D), jnp.int32),
            pltpu.SemaphoreType.DMA,
            pltpu.SemaphoreType.DMA,
            pltpu.SemaphoreType.DMA,
            pltpu.SemaphoreType.DMA,
        ],
    )
    def k(td_h, ts_h, dst_h, src_h, g_h, idx_d, idx_s,
          bd0, bs0, bd1, bs1, gsem0, gsem1, wsem0, wsem1):
        wid = lax.axis_index("s") * NC + lax.axis_index("c")
        nloc = jnp.where(wid < rem, cpw + 1, cpw)
        bufs = ((bd0, bs0), (bd1, bs1))
        gsems = (gsem0, gsem1)
        wsems = (wsem0, wsem1)

        first = wid * cpw * KG
        pltpu.sync_copy(dst_h.at[pl.ds(first, cpw * KG)],
                        idx_d.at[pl.ds(0, cpw * KG)])
        pltpu.sync_copy(src_h.at[pl.ds(first, cpw * KG)],
                        idx_s.at[pl.ds(0, cpw * KG)])

        @pl.when(wid < rem)
        def _extra():
            eb = (cpw * NW + wid) * KG
            pltpu.sync_copy(dst_h.at[pl.ds(eb, KG)],
                            idx_d.at[pl.ds(cpw * KG, KG)])
            pltpu.sync_copy(src_h.at[pl.ds(eb, KG)],
                            idx_s.at[pl.ds(cpw * KG, KG)])

        def base_of(j):
            c = jnp.where(j < cpw, wid * cpw + j, cpw * NW + wid)
            return c * KG

        def stage_and_fire(j, slot):
            sl = pl.ds(j * KG, KG)
            pltpu.async_copy(td_h.at[idx_d.at[sl]], bufs[slot][0], gsems[slot])
            pltpu.async_copy(ts_h.at[idx_s.at[sl]], bufs[slot][1], gsems[slot])

        def wait_gathers(slot):
            z = pl.ds(0, KG)
            pltpu.make_async_copy(
                td_h.at[idx_d.at[z]], bufs[slot][0], gsems[slot]).wait()
            pltpu.make_async_copy(
                ts_h.at[idx_s.at[z]], bufs[slot][1], gsems[slot]).wait()

        def drain_writeout(j, slot):
            pltpu.make_async_copy(
                bufs[slot][0], g_h.at[pl.ds(base_of(j), KG)],
                wsems[slot]).wait()

        stage_and_fire(0, 0)

        def pair(j2, carry):
            for b in range(2):
                j = j2 * 2 + b
                slot = b
                other = 1 - b

                @pl.when(j < nloc)
                def _step():
                    wait_gathers(slot)

                    @pl.when(j + 1 < nloc)
                    def _fire_next():
                        @pl.when(j >= 1)
                        def _drain_prev():
                            drain_writeout(j - 1, other)

                        stage_and_fire(j + 1, other)

                    bd, bs = bufs[slot]

                    @plsc.parallel_loop(0, KG, unroll=2)
                    def _add(r):
                        for t in range(D // 16):
                            sl = pl.ds(t * 16, 16)
                            a = plsc.bitcast(bd[r, sl], jnp.bfloat16)
                            b = plsc.bitcast(bs[r, sl], jnp.bfloat16)
                            bd[r, sl] = plsc.bitcast(a + b, jnp.int32)

                    pltpu.async_copy(
                        bufs[slot][0], g_h.at[pl.ds(base_of(j), KG)],
                        wsems[slot])
            return carry

        lax.fori_loop(0, (nloc + 1) // 2, pair, 0)

        last_even = (nloc - 1) % 2 == 0

        @pl.when((nloc >= 1) & last_even)
        def _drain_a():
            drain_writeout(nloc - 1, 0)

        @pl.when((nloc >= 1) & jnp.logical_not(last_even))
        def _drain_b():
            drain_writeout(nloc - 1, 1)

        @pl.when((nloc >= 2) & last_even)
        def _drain_c():
            drain_writeout(nloc - 2, 1)

        @pl.when((nloc >= 2) & jnp.logical_not(last_even))
        def _drain_d():
            drain_writeout(nloc - 2, 0)

    return k(td, ts, dst, src)


def _scatter_call(m0, m1, dst):
    ne = dst.shape[0]
    C = ne // KS

    @functools.partial(
        pl.kernel,
        out_type=(jax.ShapeDtypeStruct((N, 128), jnp.float32),
                  jax.ShapeDtypeStruct((N, 128), jnp.float32)),
        mesh=_sc_mesh(),
        scratch_types=[
            pltpu.VMEM((KS,), jnp.int32),
            pltpu.VMEM((KS,), jnp.int32),
            pltpu.VMEM((KS, 128), jnp.float32),
            pltpu.VMEM((KS, 128), jnp.float32),
            pltpu.VMEM((RW, 128), jnp.float32),
            pltpu.VMEM_SHARED((N, 128), jnp.float32),
            pltpu.SemaphoreType.DMA,
            pltpu.SemaphoreType.DMA,
            pltpu.SemaphoreType.DMA,
            pltpu.SemaphoreType.DMA,
        ],
    )
    def k(m0_h, m1_h, dst_h, agg0_h, agg1_h, idx0, idx1, mb0, mb1, obuf, acc,
          lsem0, lsem1, ssem0, ssem1):
        cid = lax.axis_index("c")
        sid = lax.axis_index("s")
        idx = (idx0, idx1)
        mbuf = (mb0, mb1)
        lsems = (lsem0, lsem1)
        ssems = (ssem0, ssem1)

        def zrow(r, carry):
            def zlane(j, c2):
                obuf[r, pl.ds(j * 16, 16)] = jnp.zeros((16,), jnp.float32)
                return c2
            return lax.fori_loop(0, 128 // 16, zlane, carry)

        lax.fori_loop(0, RW, zrow, 0)
        nw = (CW - sid + NS - 1) // NS

        def zchunk(j, carry):
            t = sid + j * NS
            pltpu.sync_copy(obuf, acc.at[pl.ds(t * RW, RW)])
            return carry

        lax.fori_loop(0, nw, zchunk, 0)
        plsc.subcore_barrier()

        nloc = (C - sid + NS - 1) // NS

        def base_of(j):
            return (sid + j * NS) * KS

        def fire_loads(j, slot):
            base = base_of(j)
            pltpu.async_copy(dst_h.at[pl.ds(base, KS)], idx[slot], lsems[slot])

            @pl.when(cid == 0)
            def _l0():
                pltpu.async_copy(m0_h.at[pl.ds(base, KS)], mbuf[slot],
                                 lsems[slot])

            @pl.when(cid == 1)
            def _l1():
                pltpu.async_copy(m1_h.at[pl.ds(base, KS)], mbuf[slot],
                                 lsems[slot])

        def wait_loads(j, slot):
            base = base_of(j)
            pltpu.make_async_copy(
                dst_h.at[pl.ds(base, KS)], idx[slot], lsems[slot]).wait()
            pltpu.make_async_copy(
                m0_h.at[pl.ds(base, KS)], mbuf[slot], lsems[slot]).wait()

        def drain_scatter(slot):
            pltpu.make_async_copy(mbuf[slot], acc.at[idx[slot]],
                                  ssems[slot]).wait()

        fire_loads(0, 0)

        def pair(j2, carry):
            for b in range(2):
                j = j2 * 2 + b
                slot = b
                other = 1 - b

                @pl.when(j < nloc)
                def _step():
                    wait_loads(j, slot)

                    @pl.when(j + 1 < nloc)
                    def _fire_next():
                        @pl.when(j >= 1)
                        def _drain_prev():
                            drain_scatter(other)

                        fire_loads(j + 1, other)

                    pltpu.async_copy(mbuf[slot], acc.at[idx[slot]],
                                     ssems[slot], add=True)
            return carry

        lax.fori_loop(0, (nloc + 1) // 2, pair, 0)

        last_even = (nloc - 1) % 2 == 0

        @pl.when((nloc >= 1) & last_even)
        def _drain_a():
            drain_scatter(0)

        @pl.when((nloc >= 1) & jnp.logical_not(last_even))
        def _drain_b():
            drain_scatter(1)

        @pl.when((nloc >= 2) & last_even)
        def _drain_c():
            drain_scatter(1)

        @pl.when((nloc >= 2) & jnp.logical_not(last_even))
        def _drain_d():
            drain_scatter(0)

        plsc.subcore_barrier()

        def ochunk(j, carry):
            r0 = (sid + j * NS) * RW
            pltpu.sync_copy(acc.at[pl.ds(r0, RW)], obuf)

            @pl.when(cid == 0)
            def _s0():
                pltpu.sync_copy(obuf, agg0_h.at[pl.ds(r0, RW)])

            @pl.when(cid == 1)
            def _s1():
                pltpu.sync_copy(obuf, agg1_h.at[pl.ds(r0, RW)])

            return carry

        lax.fori_loop(0, nw, ochunk, 0)

    return k(m0, m1, dst)


def kernel(x, edge_index, edge_attr, energies, batch, emb, We1, We2, Wsk,
           Wf, Ws, Wfe1, Wfe2, Wfc1, Wfc2, be1, be2, bfe1, bfe2, bfc1, bfc2):
    src = edge_index[0].astype(jnp.int32)
    dst = edge_index[1].astype(jnp.int32)
    x3 = x.astype(jnp.int32).reshape(N // BN, 1, BN)
    b3 = batch.astype(jnp.int32).reshape(N // BN, 1, BN)

    wd = []
    wsrc = []
    wedge = []
    for i in range(3):
        wd.append(jnp.concatenate([Wf[i, :D, :], Ws[i, :D, :]], axis=1))
        wsrc.append(jnp.concatenate([Wf[i, D:2 * D, :], Ws[i, D:2 * D, :]], axis=1))
        wedge.append(jnp.concatenate([Wf[i, 2 * D:, :], Ws[i, 2 * D:, :]], axis=1))

    be1r = be1.reshape(1, 128)
    be2r = be2.reshape(1, D)

    EH = E // 2
    dsth = (dst[:EH], dst[EH:])
    srch = (src[:EH], src[EH:])
    eah = (edge_attr[:EH], edge_attr[EH:])

    td, ts, s0, s1 = _node0_call(x3, emb, wd[0], wsrc[0], Wsk[0])
    aggs = None
    for i in range(3):
        mh = []
        aggs = []
        for h in range(2):
            g = _gatheradd_call(td, ts, dsth[h], srch[h])
            mh.append(_edge_call(eah[h], g, We1, be1r, We2, be2r, wedge[i]))
        for h in range(2):
            aggs.append(_scatter_call(mh[h][0], mh[h][1], dsth[h]))
        (a0a, a1a), (a0b, a1b) = aggs
        if i < 2:
            td, ts, s0n, s1n = _node12_call(
                a0a, a1a, a0b, a1b, s0, s1, wd[i + 1], wsrc[i + 1], Wsk[i + 1])
            s0, s1 = s0n, s1n

    (a0a, a1a), (a0b, a1b) = aggs
    sump, maxp, cnt = _pool_call(a0a, a1a, a0b, a1b, s0, s1, b3)
    out = _head_call(
        sump, maxp, cnt, energies, Wfe1, bfe1.reshape(1, D),
        Wfe2, bfe2.reshape(1, 128), Wfc1, bfc1.reshape(1, 1024),
        Wfc2, bfc2.reshape(1, 804))
    return out.reshape(G, 4, 201)
